# Initial kernel scaffold; baseline (speedup 1.0000x reference)
import functools

import jax
import jax.numpy as jnp
from jax import lax
from jax.experimental import pallas as pl
from jax.experimental.pallas import tpu as pltpu
from jax.experimental.pallas import tpu_sc as plsc

_NS, _NC, _NL = 50000, 10000, 10000
_DIN, _DH = 128, 64
_EU, _ET = 800000, 320000
_W = 80

_EU_PAD, _ET_PAD = 800768, 321536
_CPT_U, _CPT_T = 391, 157
_HALF_U, _HALF_T = _NS // 2, _NL // 2
_ACC_ROWS = 25088


def _sc_body(hc_ref, su_ref, du_ref, st_ref, dt_ref, outu_ref, outt_ref,
             acc, zbuf, srcbuf, dstbuf, dloc, rows, semg):
    c = lax.axis_index("c")
    s = lax.axis_index("s")

    def _zrow(r, carry):
        for k in range(_W // 16):
            zbuf[r, pl.ds(k * 16, 16)] = jnp.zeros((16,), jnp.float32)
        return carry
    lax.fori_loop(0, 128, _zrow, 0)

    def _zero_region(n128, tail):
        r0 = s * (n128 * 128 + tail)
        def _za(k, carry):
            pltpu.sync_copy(zbuf, acc.at[pl.ds(r0 + k * 128, 128)])
            return carry
        lax.fori_loop(0, n128, _za, 0)
        if tail:
            pltpu.sync_copy(zbuf.at[pl.ds(0, tail)],
                            acc.at[pl.ds(r0 + n128 * 128, tail)])

    def _scatter_phase(src_ref, dst_ref, n_chunks_tile, half, trash):
        lo = c * half
        hi = lo + half
        c0 = s * n_chunks_tile
        def _body(j, carry):
            eb = (c0 + j) * 128
            pltpu.sync_copy(src_ref.at[pl.ds(eb, 128)], srcbuf)
            pltpu.sync_copy(dst_ref.at[pl.ds(eb, 128)], dstbuf)
            for i in range(8):
                d = dstbuf[pl.ds(i * 16, 16)]
                ok = (d >= lo) & (d < hi)
                dloc[0, pl.ds(i * 16, 16)] = jnp.where(ok, d - lo, trash)
            pltpu.async_copy(hc_ref.at[srcbuf], rows, semg).wait()
            pltpu.sync_copy(rows, acc.at[dloc.at[0]], add=True)
            return carry
        lax.fori_loop(0, n_chunks_tile, _body, 0)

    def _writeout(out_ref, n128, tail):
        r0 = s * (n128 * 128 + tail)
        def _wo(k, carry):
            pltpu.sync_copy(acc.at[pl.ds(r0 + k * 128, 128)],
                            out_ref.at[c, pl.ds(r0 + k * 128, 128)])
            return carry
        lax.fori_loop(0, n128, _wo, 0)
        if tail:
            pltpu.sync_copy(acc.at[pl.ds(r0 + n128 * 128, tail)],
                            out_ref.at[c, pl.ds(r0 + n128 * 128, tail)])

    _zero_region(12, 32)
    plsc.subcore_barrier()
    _scatter_phase(su_ref, du_ref, _CPT_U, _HALF_U, _HALF_U)
    plsc.subcore_barrier()
    _writeout(outu_ref, 12, 32)
    plsc.subcore_barrier()
    _zero_region(2, 64)
    plsc.subcore_barrier()
    _scatter_phase(st_ref, dt_ref, _CPT_T, _HALF_T, _HALF_T)
    plsc.subcore_barrier()
    _writeout(outt_ref, 2, 64)


_sc_agg = functools.partial(
    pl.kernel,
    out_type=(jax.ShapeDtypeStruct((2, _ACC_ROWS, _W), jnp.float32),
              jax.ShapeDtypeStruct((2, 5120, _W), jnp.float32)),
    mesh=plsc.VectorSubcoreMesh(core_axis_name="c", subcore_axis_name="s",
                                num_cores=2, num_subcores=16),
    scratch_types=[
        pltpu.VMEM_SHARED((_ACC_ROWS, _W), jnp.float32),
        pltpu.VMEM((128, _W), jnp.float32),
        pltpu.VMEM((128,), jnp.int32),
        pltpu.VMEM((128,), jnp.int32),
        pltpu.VMEM((1, 128), jnp.int32),
        pltpu.VMEM((128, _W), jnp.float32),
        pltpu.SemaphoreType.DMA,
    ],
)(_sc_body)


def _aug_body(x_ref, w_ref, b_ref, o_ref):
    h = jnp.dot(x_ref[...], w_ref[...],
                preferred_element_type=jnp.float32) + b_ref[...]
    one = jnp.ones((h.shape[0], 1), jnp.float32)
    pad = jnp.zeros((h.shape[0], _W - _DH - 1), jnp.float32)
    o_ref[...] = jnp.concatenate([h, one, pad], axis=1)


def _project_aug(x, w, b):
    m, bm = x.shape[0], 2000
    return pl.pallas_call(
        _aug_body,
        grid=(m // bm,),
        in_specs=[pl.BlockSpec((bm, _DIN), lambda i: (i, 0)),
                  pl.BlockSpec((_DIN, _DH), lambda i: (0, 0)),
                  pl.BlockSpec((1, _DH), lambda i: (0, 0))],
        out_specs=pl.BlockSpec((bm, _W), lambda i: (i, 0)),
        out_shape=jax.ShapeDtypeStruct((m, _W), jnp.float32),
    )(x, w, b.reshape(1, _DH))


def _chain_body(x_ref, g_ref, wfc_ref, bfc_ref, ws_ref, wn_ref, b_ref, o_ref):
    h = jnp.dot(x_ref[...], wfc_ref[...],
                preferred_element_type=jnp.float32) + bfc_ref[...]
    agg = g_ref[:, 0:_DH]
    deg = g_ref[:, _DH:_DH + 1]
    nbr = agg / jnp.maximum(deg, 1.0)
    for i in range(3):
        z = (jnp.dot(h, ws_ref[i], preferred_element_type=jnp.float32)
             + jnp.dot(nbr, wn_ref[i], preferred_element_type=jnp.float32)
             + b_ref[i])
        h = jnp.maximum(z, 0.0) if i < 2 else z
    o_ref[...] = h


def _chain(x, agg80, wfc, bfc, ws, wn, b):
    m, bm = x.shape[0], 2000
    return pl.pallas_call(
        _chain_body,
        grid=(m // bm,),
        in_specs=[pl.BlockSpec((bm, _DIN), lambda i: (i, 0)),
                  pl.BlockSpec((bm, _W), lambda i: (i, 0)),
                  pl.BlockSpec((_DIN, _DH), lambda i: (0, 0)),
                  pl.BlockSpec((1, _DH), lambda i: (0, 0)),
                  pl.BlockSpec((3, _DH, _DH), lambda i: (0, 0, 0)),
                  pl.BlockSpec((3, _DH, _DH), lambda i: (0, 0, 0)),
                  pl.BlockSpec((3, _DH), lambda i: (0, 0))],
        out_specs=pl.BlockSpec((bm, _DH), lambda i: (i, 0)),
        out_shape=jax.ShapeDtypeStruct((m, _DH), jnp.float32),
    )(x, agg80, wfc, bfc.reshape(1, _DH), ws, wn, b)


def kernel(x_student, x_concept, x_lecture, Wfc_s, bfc_s, Wfc_c, bfc_c,
           Wfc_l, bfc_l, Ws_u, Wn_u, b_u, Ws_t, Wn_t, b_t,
           src_understands, dst_understands, src_teaches, dst_teaches):
    hc_aug = _project_aug(x_concept, Wfc_c, bfc_c)

    i32 = jnp.int32
    su = jnp.concatenate([src_understands.astype(i32),
                          jnp.zeros((_EU_PAD - _EU,), i32)])
    du = jnp.concatenate([dst_understands.astype(i32),
                          jnp.full((_EU_PAD - _EU,), _NS, i32)])
    st = jnp.concatenate([src_teaches.astype(i32),
                          jnp.zeros((_ET_PAD - _ET,), i32)])
    dt = jnp.concatenate([dst_teaches.astype(i32),
                          jnp.full((_ET_PAD - _ET,), _NL, i32)])

    out_u, out_t = _sc_agg(hc_aug, su, du, st, dt)
    agg_s = jnp.concatenate([out_u[0, :_HALF_U], out_u[1, :_HALF_U]], axis=0)
    agg_l = jnp.concatenate([out_t[0, :_HALF_T], out_t[1, :_HALF_T]], axis=0)

    hs_out = _chain(x_student, agg_s, Wfc_s, bfc_s, Ws_u, Wn_u, b_u)
    hl_out = _chain(x_lecture, agg_l, Wfc_l, bfc_l, Ws_t, Wn_t, b_t)
    return hs_out, hc_aug[:, :_DH], hl_out


# speedup vs baseline: 2.5501x; 2.5501x over previous
import functools

import jax
import jax.numpy as jnp
from jax import lax
from jax.experimental import pallas as pl
from jax.experimental.pallas import tpu as pltpu
from jax.experimental.pallas import tpu_sc as plsc

_NS, _NC, _NL = 50000, 10000, 10000
_DIN, _DH = 128, 64
_EU, _ET = 800000, 320000
_W = 80

_CHUNK = 64
_EU_PAD, _ET_PAD = 800768, 321536
_CPT_U, _CPT_T = 782, 314
_HALF_U, _HALF_T = _NS // 2, _NL // 2
_ACC_ROWS = 25008


def _sc_body(hc_ref, su_ref, du_ref, st_ref, dt_ref, outu_ref, outt_ref,
             acc, srcbuf, dstbuf, dloc, rows, semg):
    c = lax.axis_index("c")
    s = lax.axis_index("s")

    def _zero_rows():
        def _zrow(r, carry):
            for k in range(_W // 16):
                rows[r, pl.ds(k * 16, 16)] = jnp.zeros((16,), jnp.float32)
            return carry
        lax.fori_loop(0, _CHUNK, _zrow, 0)

    def _zero_region(n64, tail):
        r0 = s * (n64 * _CHUNK + tail)
        def _za(k, carry):
            pltpu.sync_copy(rows, acc.at[pl.ds(r0 + k * _CHUNK, _CHUNK)])
            return carry
        lax.fori_loop(0, n64, _za, 0)
        pltpu.sync_copy(rows.at[pl.ds(0, tail)],
                        acc.at[pl.ds(r0 + n64 * _CHUNK, tail)])

    def _scatter_phase(src_ref, dst_ref, n_chunks_tile, half, trash):
        lo = c * half
        hi = lo + half
        c0 = s * n_chunks_tile
        def _body(j, carry):
            eb = (c0 + j) * _CHUNK
            pltpu.sync_copy(src_ref.at[pl.ds(eb, _CHUNK)], srcbuf)
            pltpu.sync_copy(dst_ref.at[pl.ds(eb, _CHUNK)], dstbuf)
            for i in range(_CHUNK // 16):
                d = dstbuf[pl.ds(i * 16, 16)]
                ok = (d >= lo) & (d < hi)
                dloc[0, pl.ds(i * 16, 16)] = jnp.where(ok, d - lo, trash)
            pltpu.async_copy(hc_ref.at[srcbuf], rows, semg).wait()
            pltpu.sync_copy(rows, acc.at[dloc.at[0]], add=True)
            return carry
        lax.fori_loop(0, n_chunks_tile, _body, 0)

    def _writeout(out_ref, n64, tail):
        r0 = s * (n64 * _CHUNK + tail)
        def _wo(k, carry):
            pltpu.sync_copy(acc.at[pl.ds(r0 + k * _CHUNK, _CHUNK)],
                            out_ref.at[c, pl.ds(r0 + k * _CHUNK, _CHUNK)])
            return carry
        lax.fori_loop(0, n64, _wo, 0)
        pltpu.sync_copy(acc.at[pl.ds(r0 + n64 * _CHUNK, tail)],
                        out_ref.at[c, pl.ds(r0 + n64 * _CHUNK, tail)])

    _zero_rows()
    _zero_region(24, 27)
    plsc.subcore_barrier()
    _scatter_phase(su_ref, du_ref, _CPT_U, _HALF_U, _HALF_U)
    plsc.subcore_barrier()
    _writeout(outu_ref, 24, 27)
    plsc.subcore_barrier()
    _zero_rows()
    _zero_region(4, 57)
    plsc.subcore_barrier()
    _scatter_phase(st_ref, dt_ref, _CPT_T, _HALF_T, _HALF_T)
    plsc.subcore_barrier()
    _writeout(outt_ref, 4, 57)


_sc_agg = functools.partial(
    pl.kernel,
    out_type=(jax.ShapeDtypeStruct((2, _ACC_ROWS, _W), jnp.float32),
              jax.ShapeDtypeStruct((2, 5008, _W), jnp.float32)),
    mesh=plsc.VectorSubcoreMesh(core_axis_name="c", subcore_axis_name="s",
                                num_cores=2, num_subcores=16),
    compiler_params=pltpu.CompilerParams(use_tc_tiling_on_sc=False),
    scratch_types=[
        pltpu.VMEM_SHARED((_ACC_ROWS, _W), jnp.float32),
        pltpu.VMEM((_CHUNK,), jnp.int32),
        pltpu.VMEM((_CHUNK,), jnp.int32),
        pltpu.VMEM((1, _CHUNK), jnp.int32),
        pltpu.VMEM((_CHUNK, _W), jnp.float32),
        pltpu.SemaphoreType.DMA,
    ],
)(_sc_body)


def _aug_body(x_ref, w_ref, b_ref, o_ref):
    h = jnp.dot(x_ref[...], w_ref[...],
                preferred_element_type=jnp.float32) + b_ref[...]
    one = jnp.ones((h.shape[0], 1), jnp.float32)
    pad = jnp.zeros((h.shape[0], _W - _DH - 1), jnp.float32)
    o_ref[...] = jnp.concatenate([h, one, pad], axis=1)


def _project_aug(x, w, b):
    m, bm = x.shape[0], 2000
    return pl.pallas_call(
        _aug_body,
        grid=(m // bm,),
        in_specs=[pl.BlockSpec((bm, _DIN), lambda i: (i, 0)),
                  pl.BlockSpec((_DIN, _DH), lambda i: (0, 0)),
                  pl.BlockSpec((1, _DH), lambda i: (0, 0))],
        out_specs=pl.BlockSpec((bm, _W), lambda i: (i, 0)),
        out_shape=jax.ShapeDtypeStruct((m, _W), jnp.float32),
    )(x, w, b.reshape(1, _DH))


def _chain_body(x_ref, g_ref, wfc_ref, bfc_ref, ws_ref, wn_ref, b_ref, o_ref):
    h = jnp.dot(x_ref[...], wfc_ref[...],
                preferred_element_type=jnp.float32) + bfc_ref[...]
    agg = g_ref[:, 0:_DH]
    deg = g_ref[:, _DH:_DH + 1]
    nbr = agg / jnp.maximum(deg, 1.0)
    for i in range(3):
        z = (jnp.dot(h, ws_ref[i], preferred_element_type=jnp.float32)
             + jnp.dot(nbr, wn_ref[i], preferred_element_type=jnp.float32)
             + b_ref[i])
        h = jnp.maximum(z, 0.0) if i < 2 else z
    o_ref[...] = h


def _chain(x, agg80, wfc, bfc, ws, wn, b):
    m, bm = x.shape[0], 2000
    return pl.pallas_call(
        _chain_body,
        grid=(m // bm,),
        in_specs=[pl.BlockSpec((bm, _DIN), lambda i: (i, 0)),
                  pl.BlockSpec((bm, _W), lambda i: (i, 0)),
                  pl.BlockSpec((_DIN, _DH), lambda i: (0, 0)),
                  pl.BlockSpec((1, _DH), lambda i: (0, 0)),
                  pl.BlockSpec((3, _DH, _DH), lambda i: (0, 0, 0)),
                  pl.BlockSpec((3, _DH, _DH), lambda i: (0, 0, 0)),
                  pl.BlockSpec((3, _DH), lambda i: (0, 0))],
        out_specs=pl.BlockSpec((bm, _DH), lambda i: (i, 0)),
        out_shape=jax.ShapeDtypeStruct((m, _DH), jnp.float32),
    )(x, agg80, wfc, bfc.reshape(1, _DH), ws, wn, b)


def kernel(x_student, x_concept, x_lecture, Wfc_s, bfc_s, Wfc_c, bfc_c,
           Wfc_l, bfc_l, Ws_u, Wn_u, b_u, Ws_t, Wn_t, b_t,
           src_understands, dst_understands, src_teaches, dst_teaches):
    hc_aug = _project_aug(x_concept, Wfc_c, bfc_c)

    i32 = jnp.int32
    su = jnp.concatenate([src_understands.astype(i32),
                          jnp.zeros((_EU_PAD - _EU,), i32)])
    du = jnp.concatenate([dst_understands.astype(i32),
                          jnp.full((_EU_PAD - _EU,), _NS, i32)])
    st = jnp.concatenate([src_teaches.astype(i32),
                          jnp.zeros((_ET_PAD - _ET,), i32)])
    dt = jnp.concatenate([dst_teaches.astype(i32),
                          jnp.full((_ET_PAD - _ET,), _NL, i32)])

    out_u, out_t = _sc_agg(hc_aug, su, du, st, dt)
    agg_s = jnp.concatenate([out_u[0, :_HALF_U], out_u[1, :_HALF_U]], axis=0)
    agg_l = jnp.concatenate([out_t[0, :_HALF_T], out_t[1, :_HALF_T]], axis=0)

    hs_out = _chain(x_student, agg_s, Wfc_s, bfc_s, Ws_u, Wn_u, b_u)
    hl_out = _chain(x_lecture, agg_l, Wfc_l, bfc_l, Ws_t, Wn_t, b_t)
    return hs_out, hc_aug[:, :_DH], hl_out


# speedup vs baseline: 3.5740x; 1.4015x over previous
import functools

import jax
import jax.numpy as jnp
from jax import lax
from jax.experimental import pallas as pl
from jax.experimental.pallas import tpu as pltpu
from jax.experimental.pallas import tpu_sc as plsc

_NS, _NC, _NL = 50000, 10000, 10000
_DIN, _DH = 128, 64
_EU, _ET = 800000, 320000

_EU_PAD, _ET_PAD = 800768, 321536
_CPT_U, _CPT_T = 391, 157
_SUP = 23
_HALF_U, _HALF_T = _NS // 2, _NL // 2
_ROWS_U = 25008
_ROWS_T = 5008


def _feat_body(hc_ref, su_ref, du_ref, st_ref, dt_ref, outu_ref, outt_ref,
               acc, ssrc, sdst, dloc, rows_a, rows_b,
               semg_a, semg_b, sems_a, sems_b):
    c = lax.axis_index("c")
    s = lax.axis_index("s")
    rows = (rows_a, rows_b)
    semg = (semg_a, semg_b)
    sems = (sems_a, sems_b)

    def _zero_rows_a():
        def _zr(r, carry):
            for k in range(_DH // 16):
                rows_a[r, pl.ds(k * 16, 16)] = jnp.zeros((16,), jnp.float32)
            return carry
        lax.fori_loop(0, 128, _zr, 0)

    def _zero_region(n128, tail):
        r0 = s * (n128 * 128 + tail)
        def _za(k, carry):
            pltpu.sync_copy(rows_a, acc.at[pl.ds(r0 + k * 128, 128)])
            return carry
        lax.fori_loop(0, n128, _za, 0)
        pltpu.sync_copy(rows_a.at[pl.ds(0, tail)],
                        acc.at[pl.ds(r0 + n128 * 128, tail)])

    def _dloc_of(k, slot, lo, hi, trash):
        for i in range(8):
            d = sdst[k, pl.ds(i * 16, 16)]
            ok = (d >= lo) & (d < hi)
            dloc[slot, pl.ds(i * 16, 16)] = jnp.where(ok, d - lo, trash)

    def _super(src2_ref, dst2_ref, chunk0, nk, lo, hi, trash):
        pltpu.sync_copy(src2_ref.at[pl.ds(chunk0, nk)], ssrc.at[pl.ds(0, nk)])
        pltpu.sync_copy(dst2_ref.at[pl.ds(chunk0, nk)], sdst.at[pl.ds(0, nk)])
        _dloc_of(0, 0, lo, hi, trash)
        gd = [None] * nk
        sd = [None] * nk
        gd[0] = pltpu.async_copy(hc_ref.at[ssrc.at[0]], rows[0], semg[0])
        for j in range(nk):
            cur, nxt = j % 2, (j + 1) % 2
            if j >= 1:
                sd[j - 1].wait()
            if j + 1 < nk:
                _dloc_of(j + 1, nxt, lo, hi, trash)
                gd[j + 1] = pltpu.async_copy(hc_ref.at[ssrc.at[j + 1]],
                                             rows[nxt], semg[nxt])
            gd[j].wait()
            sd[j] = pltpu.async_copy(rows[cur], acc.at[dloc.at[cur]],
                                     sems[cur], add=True)
        sd[nk - 1].wait()

    def _phase(src2_ref, dst2_ref, n_sup, chunks_tile, half, trash):
        lo = c * half
        hi = lo + half
        base = s * chunks_tile
        def _ps(m, carry):
            _super(src2_ref, dst2_ref, base + m * _SUP, _SUP, lo, hi, trash)
            return carry
        lax.fori_loop(0, n_sup, _ps, 0)
        rem = chunks_tile - n_sup * _SUP
        if rem:
            _super(src2_ref, dst2_ref, base + n_sup * _SUP, rem, lo, hi, trash)

    def _writeout(out_ref, n128, tail):
        r0 = s * (n128 * 128 + tail)
        def _wo(k, carry):
            pltpu.sync_copy(acc.at[pl.ds(r0 + k * 128, 128)],
                            out_ref.at[c, pl.ds(r0 + k * 128, 128)])
            return carry
        lax.fori_loop(0, n128, _wo, 0)
        pltpu.sync_copy(acc.at[pl.ds(r0 + n128 * 128, tail)],
                        out_ref.at[c, pl.ds(r0 + n128 * 128, tail)])

    _zero_rows_a()
    _zero_region(12, 27)
    plsc.subcore_barrier()
    _phase(su_ref, du_ref, 17, _CPT_U, _HALF_U, _HALF_U)
    plsc.subcore_barrier()
    _writeout(outu_ref, 12, 27)
    plsc.subcore_barrier()
    _zero_rows_a()
    _zero_region(2, 57)
    plsc.subcore_barrier()
    _phase(st_ref, dt_ref, 6, _CPT_T, _HALF_T, _HALF_T)
    plsc.subcore_barrier()
    _writeout(outt_ref, 2, 57)


_sc_feat = functools.partial(
    pl.kernel,
    out_type=(jax.ShapeDtypeStruct((2, _ROWS_U, _DH), jnp.float32),
              jax.ShapeDtypeStruct((2, _ROWS_T, _DH), jnp.float32)),
    mesh=plsc.VectorSubcoreMesh(core_axis_name="c", subcore_axis_name="s",
                                num_cores=2, num_subcores=16),
    compiler_params=pltpu.CompilerParams(use_tc_tiling_on_sc=False),
    scratch_types=[
        pltpu.VMEM_SHARED((_ROWS_U, _DH), jnp.float32),
        pltpu.VMEM((_SUP, 128), jnp.int32),
        pltpu.VMEM((_SUP, 128), jnp.int32),
        pltpu.VMEM((2, 128), jnp.int32),
        pltpu.VMEM((128, _DH), jnp.float32),
        pltpu.VMEM((128, _DH), jnp.float32),
        pltpu.SemaphoreType.DMA, pltpu.SemaphoreType.DMA,
        pltpu.SemaphoreType.DMA, pltpu.SemaphoreType.DMA,
    ],
)(_feat_body)


def _deg_body(du_ref, dt_ref, outu_ref, outt_ref,
              dacc, dstb, dloc, ones, zb, sems_a, sems_b):
    c = lax.axis_index("c")
    s = lax.axis_index("s")
    sems = (sems_a, sems_b)

    def _fill(buf, vec):
        def _fr(r, carry):
            buf[r, pl.ds(0, 16)] = vec
            return carry
        lax.fori_loop(0, 128, _fr, 0)

    def _zero_region(n128, tail):
        r0 = s * (n128 * 128 + tail)
        def _za(k, carry):
            pltpu.sync_copy(zb, dacc.at[pl.ds(r0 + k * 128, 128)])
            return carry
        lax.fori_loop(0, n128, _za, 0)
        pltpu.sync_copy(zb.at[pl.ds(0, tail)],
                        dacc.at[pl.ds(r0 + n128 * 128, tail)])

    def _dloc_of(ch, slot, lo, hi, trash):
        for i in range(8):
            d = dstb[pl.ds(ch * 128 + i * 16, 16)]
            ok = (d >= lo) & (d < hi)
            dloc[slot, pl.ds(i * 16, 16)] = jnp.where(ok, d - lo, trash)

    def _scat(slot):
        return pltpu.async_copy(ones, dacc.at[dloc.at[slot]], sems[slot],
                                add=True)

    def _drain(slot):
        pltpu.make_async_copy(ones, dacc.at[dloc.at[slot]], sems[slot]).wait()

    def _phase(d_ref, n_edges_tile, n_chunks, n_pairs, half, trash):
        lo = c * half
        hi = lo + half
        pltpu.sync_copy(d_ref.at[pl.ds(s * n_edges_tile, n_edges_tile)],
                        dstb.at[pl.ds(0, n_edges_tile)])
        _dloc_of(0, 0, lo, hi, trash)
        _scat(0)
        def _pair(g, carry):
            j0 = 2 * g + 1
            _drain(0)
            _dloc_of(j0, 1, lo, hi, trash)
            _scat(1)
            _drain(1)
            _dloc_of(j0 + 1, 0, lo, hi, trash)
            _scat(0)
            return carry
        lax.fori_loop(0, n_pairs, _pair, 0)
        _drain(0)

    def _writeout(out_ref, n128, tail):
        r0 = s * (n128 * 128 + tail)
        def _wo(k, carry):
            pltpu.sync_copy(dacc.at[pl.ds(r0 + k * 128, 128)],
                            out_ref.at[c, pl.ds(r0 + k * 128, 128)])
            return carry
        lax.fori_loop(0, n128, _wo, 0)
        pltpu.sync_copy(dacc.at[pl.ds(r0 + n128 * 128, tail)],
                        out_ref.at[c, pl.ds(r0 + n128 * 128, tail)])

    one_row = jnp.where(jnp.arange(16, dtype=jnp.int32) == 0,
                        jnp.float32(1), jnp.float32(0))
    _fill(zb, jnp.zeros((16,), jnp.float32))
    _fill(ones, one_row)
    _zero_region(12, 27)
    plsc.subcore_barrier()
    _phase(du_ref, _EU_PAD // 16, _CPT_U, (_CPT_U - 1) // 2, _HALF_U, _HALF_U)
    plsc.subcore_barrier()
    _writeout(outu_ref, 12, 27)
    plsc.subcore_barrier()
    _zero_region(2, 57)
    plsc.subcore_barrier()
    _phase(dt_ref, _ET_PAD // 16, _CPT_T, (_CPT_T - 1) // 2, _HALF_T, _HALF_T)
    plsc.subcore_barrier()
    _writeout(outt_ref, 2, 57)


_sc_deg = functools.partial(
    pl.kernel,
    out_type=(jax.ShapeDtypeStruct((2, _ROWS_U, 16), jnp.float32),
              jax.ShapeDtypeStruct((2, _ROWS_T, 16), jnp.float32)),
    mesh=plsc.VectorSubcoreMesh(core_axis_name="c", subcore_axis_name="s",
                                num_cores=2, num_subcores=16),
    compiler_params=pltpu.CompilerParams(use_tc_tiling_on_sc=False),
    scratch_types=[
        pltpu.VMEM_SHARED((_ROWS_U, 16), jnp.float32),
        pltpu.VMEM((_EU_PAD // 16,), jnp.int32),
        pltpu.VMEM((2, 128), jnp.int32),
        pltpu.VMEM((128, 16), jnp.float32),
        pltpu.VMEM((128, 16), jnp.float32),
        pltpu.SemaphoreType.DMA, pltpu.SemaphoreType.DMA,
    ],
)(_deg_body)


def _proj_body(x_ref, w_ref, b_ref, o_ref):
    o_ref[...] = jnp.dot(x_ref[...], w_ref[...],
                         preferred_element_type=jnp.float32) + b_ref[...]


def _project(x, w, b):
    m, bm = x.shape[0], 2000
    return pl.pallas_call(
        _proj_body,
        grid=(m // bm,),
        in_specs=[pl.BlockSpec((bm, _DIN), lambda i: (i, 0)),
                  pl.BlockSpec((_DIN, _DH), lambda i: (0, 0)),
                  pl.BlockSpec((1, _DH), lambda i: (0, 0))],
        out_specs=pl.BlockSpec((bm, _DH), lambda i: (i, 0)),
        out_shape=jax.ShapeDtypeStruct((m, _DH), jnp.float32),
    )(x, w, b.reshape(1, _DH))


def _chain_body(x_ref, g_ref, d_ref, wfc_ref, bfc_ref, ws_ref, wn_ref, b_ref,
                o_ref):
    h = jnp.dot(x_ref[...], wfc_ref[...],
                preferred_element_type=jnp.float32) + bfc_ref[...]
    nbr = g_ref[...] / jnp.maximum(d_ref[:, 0:1], 1.0)
    for i in range(3):
        z = (jnp.dot(h, ws_ref[i], preferred_element_type=jnp.float32)
             + jnp.dot(nbr, wn_ref[i], preferred_element_type=jnp.float32)
             + b_ref[i])
        h = jnp.maximum(z, 0.0) if i < 2 else z
    o_ref[...] = h


def _chain(x, agg, deg, wfc, bfc, ws, wn, b):
    m, bm = x.shape[0], 2000
    return pl.pallas_call(
        _chain_body,
        grid=(m // bm,),
        in_specs=[pl.BlockSpec((bm, _DIN), lambda i: (i, 0)),
                  pl.BlockSpec((bm, _DH), lambda i: (i, 0)),
                  pl.BlockSpec((bm, 16), lambda i: (i, 0)),
                  pl.BlockSpec((_DIN, _DH), lambda i: (0, 0)),
                  pl.BlockSpec((1, _DH), lambda i: (0, 0)),
                  pl.BlockSpec((3, _DH, _DH), lambda i: (0, 0, 0)),
                  pl.BlockSpec((3, _DH, _DH), lambda i: (0, 0, 0)),
                  pl.BlockSpec((3, _DH), lambda i: (0, 0))],
        out_specs=pl.BlockSpec((bm, _DH), lambda i: (i, 0)),
        out_shape=jax.ShapeDtypeStruct((m, _DH), jnp.float32),
    )(x, agg, deg, wfc, bfc.reshape(1, _DH), ws, wn, b)


def kernel(x_student, x_concept, x_lecture, Wfc_s, bfc_s, Wfc_c, bfc_c,
           Wfc_l, bfc_l, Ws_u, Wn_u, b_u, Ws_t, Wn_t, b_t,
           src_understands, dst_understands, src_teaches, dst_teaches):
    i32 = jnp.int32
    su = jnp.concatenate([src_understands.astype(i32),
                          jnp.zeros((_EU_PAD - _EU,), i32)])
    du = jnp.concatenate([dst_understands.astype(i32),
                          jnp.full((_EU_PAD - _EU,), _NS, i32)])
    st = jnp.concatenate([src_teaches.astype(i32),
                          jnp.zeros((_ET_PAD - _ET,), i32)])
    dt = jnp.concatenate([dst_teaches.astype(i32),
                          jnp.full((_ET_PAD - _ET,), _NL, i32)])

    gu, gt = _sc_deg(du, dt)
    hc = _project(x_concept, Wfc_c, bfc_c)
    fu, ft = _sc_feat(hc, su.reshape(-1, 128), du.reshape(-1, 128),
                      st.reshape(-1, 128), dt.reshape(-1, 128))

    agg_s = jnp.concatenate([fu[0, :_HALF_U], fu[1, :_HALF_U]], axis=0)
    agg_l = jnp.concatenate([ft[0, :_HALF_T], ft[1, :_HALF_T]], axis=0)
    deg_s = jnp.concatenate([gu[0, :_HALF_U], gu[1, :_HALF_U]], axis=0)
    deg_l = jnp.concatenate([gt[0, :_HALF_T], gt[1, :_HALF_T]], axis=0)

    hs_out = _chain(x_student, agg_s, deg_s, Wfc_s, bfc_s, Ws_u, Wn_u, b_u)
    hl_out = _chain(x_lecture, agg_l, deg_l, Wfc_l, bfc_l, Ws_t, Wn_t, b_t)
    return hs_out, hc, hl_out


# speedup vs baseline: 5.3982x; 1.5104x over previous
import functools

import jax
import jax.numpy as jnp
from jax import lax
from jax.experimental import pallas as pl
from jax.experimental.pallas import tpu as pltpu
from jax.experimental.pallas import tpu_sc as plsc

_NS, _NC, _NL = 50000, 10000, 10000
_DIN, _DH = 128, 64
_EU, _ET = 800000, 320000

_EU_PAD, _ET_PAD = 800768, 321536
_CPT_U, _CPT_T = 391, 157
_SUP = 23
_HALF_U, _HALF_T = _NS // 2, _NL // 2
_ROWS_U = 25008
_ROWS_T = 5008


def _feat_body(hc_ref, su_ref, du_ref, st_ref, dt_ref, outu_ref, outt_ref,
               acc, ssrc, sdst, dloc, rows_a, rows_b,
               semg_a, semg_b, sems_a, sems_b):
    c = lax.axis_index("c")
    s = lax.axis_index("s")
    rows = (rows_a, rows_b)
    semg = (semg_a, semg_b)
    sems = (sems_a, sems_b)

    def _zero_rows_a():
        def _zr(r, carry):
            for k in range(_DH // 16):
                rows_a[r, pl.ds(k * 16, 16)] = jnp.zeros((16,), jnp.float32)
            return carry
        lax.fori_loop(0, 128, _zr, 0)

    def _zero_region(n128, tail):
        r0 = s * (n128 * 128 + tail)
        def _za(k, carry):
            pltpu.sync_copy(rows_a, acc.at[pl.ds(r0 + k * 128, 128)])
            return carry
        lax.fori_loop(0, n128, _za, 0)
        pltpu.sync_copy(rows_a.at[pl.ds(0, tail)],
                        acc.at[pl.ds(r0 + n128 * 128, tail)])

    def _dloc_of(k, slot, lo, hi, trash):
        for i in range(8):
            d = sdst[k, pl.ds(i * 16, 16)]
            ok = (d >= lo) & (d < hi)
            dloc[slot, pl.ds(i * 16, 16)] = jnp.where(ok, d - lo, trash)

    def _super(src2_ref, dst2_ref, chunk0, nk, lo, hi, trash):
        pltpu.sync_copy(src2_ref.at[pl.ds(chunk0, nk)], ssrc.at[pl.ds(0, nk)])
        pltpu.sync_copy(dst2_ref.at[pl.ds(chunk0, nk)], sdst.at[pl.ds(0, nk)])
        _dloc_of(0, 0, lo, hi, trash)
        gd = [None] * nk
        sd = [None] * nk
        gd[0] = pltpu.async_copy(hc_ref.at[ssrc.at[0]], rows[0], semg[0])
        for j in range(nk):
            cur, nxt = j % 2, (j + 1) % 2
            if j >= 1:
                sd[j - 1].wait()
            if j + 1 < nk:
                _dloc_of(j + 1, nxt, lo, hi, trash)
                gd[j + 1] = pltpu.async_copy(hc_ref.at[ssrc.at[j + 1]],
                                             rows[nxt], semg[nxt])
            gd[j].wait()
            sd[j] = pltpu.async_copy(rows[cur], acc.at[dloc.at[cur]],
                                     sems[cur], add=True)
        sd[nk - 1].wait()

    def _phase(src2_ref, dst2_ref, n_sup, chunks_tile, half, trash):
        lo = c * half
        hi = lo + half
        base = s * chunks_tile
        def _ps(m, carry):
            _super(src2_ref, dst2_ref, base + m * _SUP, _SUP, lo, hi, trash)
            return carry
        lax.fori_loop(0, n_sup, _ps, 0)
        rem = chunks_tile - n_sup * _SUP
        if rem:
            _super(src2_ref, dst2_ref, base + n_sup * _SUP, rem, lo, hi, trash)

    def _writeout(out_ref, n128, tail):
        r0 = s * (n128 * 128 + tail)
        def _wo(k, carry):
            pltpu.sync_copy(acc.at[pl.ds(r0 + k * 128, 128)],
                            out_ref.at[c, pl.ds(r0 + k * 128, 128)])
            return carry
        lax.fori_loop(0, n128, _wo, 0)
        pltpu.sync_copy(acc.at[pl.ds(r0 + n128 * 128, tail)],
                        out_ref.at[c, pl.ds(r0 + n128 * 128, tail)])

    _zero_rows_a()
    _zero_region(12, 27)
    plsc.subcore_barrier()
    _phase(su_ref, du_ref, 17, _CPT_U, _HALF_U, _HALF_U)
    plsc.subcore_barrier()
    _writeout(outu_ref, 12, 27)
    plsc.subcore_barrier()
    _zero_rows_a()
    _zero_region(2, 57)
    plsc.subcore_barrier()
    _phase(st_ref, dt_ref, 6, _CPT_T, _HALF_T, _HALF_T)
    plsc.subcore_barrier()
    _writeout(outt_ref, 2, 57)


_sc_feat = functools.partial(
    pl.kernel,
    out_type=(jax.ShapeDtypeStruct((2, _ROWS_U, _DH), jnp.float32),
              jax.ShapeDtypeStruct((2, _ROWS_T, _DH), jnp.float32)),
    mesh=plsc.VectorSubcoreMesh(core_axis_name="c", subcore_axis_name="s",
                                num_cores=2, num_subcores=16),
    compiler_params=pltpu.CompilerParams(use_tc_tiling_on_sc=False),
    scratch_types=[
        pltpu.VMEM_SHARED((_ROWS_U, _DH), jnp.float32),
        pltpu.VMEM((_SUP, 128), jnp.int32),
        pltpu.VMEM((_SUP, 128), jnp.int32),
        pltpu.VMEM((2, 128), jnp.int32),
        pltpu.VMEM((128, _DH), jnp.float32),
        pltpu.VMEM((128, _DH), jnp.float32),
        pltpu.SemaphoreType.DMA, pltpu.SemaphoreType.DMA,
        pltpu.SemaphoreType.DMA, pltpu.SemaphoreType.DMA,
    ],
)(_feat_body)


_EU_PAD2, _ET_PAD2 = 802816, 323584
_DCPT_U, _DCPT_T = 196, 79
_DROWS_U = 50016
_DROWS_T = 10016


def _deg_body(du_ref, dt_ref, outu_ref, outt_ref,
              dacc, dstb, dloc, ones, zb, sem0, sem1, sem2, sem3):
    c = lax.axis_index("c")
    s = lax.axis_index("s")
    sems = (sem0, sem1, sem2, sem3)

    def _fill(buf, vec):
        def _fr(r, carry):
            buf[r, pl.ds(0, 16)] = vec
            return carry
        lax.fori_loop(0, 128, _fr, 0)

    def _zero_region(n128, tail):
        r0 = s * (n128 * 128 + tail)
        def _za(k, carry):
            pltpu.sync_copy(zb, dacc.at[pl.ds(r0 + k * 128, 128)])
            return carry
        lax.fori_loop(0, n128, _za, 0)
        pltpu.sync_copy(zb.at[pl.ds(0, tail)],
                        dacc.at[pl.ds(r0 + n128 * 128, tail)])

    def _dloc_of(ch, slot, n_dst):
        for i in range(8):
            d = dstb[pl.ds(ch * 128 + i * 16, 16)]
            ok = (d >= 0) & (d < n_dst)
            dloc[slot, pl.ds(i * 16, 16)] = jnp.where(ok, d, n_dst)

    def _scat(slot):
        return pltpu.async_copy(ones, dacc.at[dloc.at[slot]], sems[slot],
                                add=True)

    def _drain(slot):
        pltpu.make_async_copy(ones, dacc.at[dloc.at[slot]], sems[slot]).wait()

    def _phase(d_ref, n_edges_tile, n4, tail_chunks, n_dst):
        base = (c * 16 + s) * n_edges_tile
        pltpu.sync_copy(d_ref.at[pl.ds(base, n_edges_tile)],
                        dstb.at[pl.ds(0, n_edges_tile)])
        for t in range(4):
            for i in range(8):
                dloc[t, pl.ds(i * 16, 16)] = jnp.full((16,), n_dst, jnp.int32)
            _scat(t)
        def _quad(g, carry):
            for t in range(4):
                _drain(t)
                _dloc_of(4 * g + t, t, n_dst)
                _scat(t)
            return carry
        lax.fori_loop(0, n4, _quad, 0)
        for t in range(tail_chunks):
            _drain(t)
            _dloc_of(4 * n4 + t, t, n_dst)
            _scat(t)
        for t in range(4):
            _drain(t)

    def _writeout(out_ref, n128, tail):
        r0 = s * (n128 * 128 + tail)
        def _wo(k, carry):
            pltpu.sync_copy(dacc.at[pl.ds(r0 + k * 128, 128)],
                            out_ref.at[c, pl.ds(r0 + k * 128, 128)])
            return carry
        lax.fori_loop(0, n128, _wo, 0)
        pltpu.sync_copy(dacc.at[pl.ds(r0 + n128 * 128, tail)],
                        out_ref.at[c, pl.ds(r0 + n128 * 128, tail)])

    one_row = jnp.where(jnp.arange(16, dtype=jnp.int32) == 0,
                        jnp.float32(1), jnp.float32(0))
    _fill(zb, jnp.zeros((16,), jnp.float32))
    _fill(ones, one_row)
    _zero_region(24, 54)
    plsc.subcore_barrier()
    _phase(du_ref, _EU_PAD2 // 32, _DCPT_U // 4, 0, _NS)
    plsc.subcore_barrier()
    _writeout(outu_ref, 24, 54)
    plsc.subcore_barrier()
    _zero_region(4, 114)
    plsc.subcore_barrier()
    _phase(dt_ref, _ET_PAD2 // 32, _DCPT_T // 4, 3, _NL)
    plsc.subcore_barrier()
    _writeout(outt_ref, 4, 114)


_sc_deg = functools.partial(
    pl.kernel,
    out_type=(jax.ShapeDtypeStruct((2, _DROWS_U, 16), jnp.float32),
              jax.ShapeDtypeStruct((2, _DROWS_T, 16), jnp.float32)),
    mesh=plsc.VectorSubcoreMesh(core_axis_name="c", subcore_axis_name="s",
                                num_cores=2, num_subcores=16),
    compiler_params=pltpu.CompilerParams(use_tc_tiling_on_sc=False),
    scratch_types=[
        pltpu.VMEM_SHARED((_DROWS_U, 16), jnp.float32),
        pltpu.VMEM((_EU_PAD2 // 32,), jnp.int32),
        pltpu.VMEM((4, 128), jnp.int32),
        pltpu.VMEM((128, 16), jnp.float32),
        pltpu.VMEM((128, 16), jnp.float32),
        pltpu.SemaphoreType.DMA, pltpu.SemaphoreType.DMA,
        pltpu.SemaphoreType.DMA, pltpu.SemaphoreType.DMA,
    ],
)(_deg_body)


def _proj_body(x_ref, w_ref, b_ref, o_ref):
    o_ref[...] = jnp.dot(x_ref[...], w_ref[...],
                         preferred_element_type=jnp.float32) + b_ref[...]


def _project(x, w, b):
    m, bm = x.shape[0], 2000
    return pl.pallas_call(
        _proj_body,
        grid=(m // bm,),
        in_specs=[pl.BlockSpec((bm, _DIN), lambda i: (i, 0)),
                  pl.BlockSpec((_DIN, _DH), lambda i: (0, 0)),
                  pl.BlockSpec((1, _DH), lambda i: (0, 0))],
        out_specs=pl.BlockSpec((bm, _DH), lambda i: (i, 0)),
        out_shape=jax.ShapeDtypeStruct((m, _DH), jnp.float32),
    )(x, w, b.reshape(1, _DH))


def _chain_body(x_ref, g_ref, d_ref, wfc_ref, bfc_ref, ws_ref, wn_ref, b_ref,
                o_ref):
    h = jnp.dot(x_ref[...], wfc_ref[...],
                preferred_element_type=jnp.float32) + bfc_ref[...]
    deg = d_ref[0, :, 0:1] + d_ref[1, :, 0:1]
    nbr = g_ref[...] / jnp.maximum(deg, 1.0)
    for i in range(3):
        z = (jnp.dot(h, ws_ref[i], preferred_element_type=jnp.float32)
             + jnp.dot(nbr, wn_ref[i], preferred_element_type=jnp.float32)
             + b_ref[i])
        h = jnp.maximum(z, 0.0) if i < 2 else z
    o_ref[...] = h


def _chain(x, agg, deg, wfc, bfc, ws, wn, b):
    m, bm = x.shape[0], 2000
    return pl.pallas_call(
        _chain_body,
        grid=(m // bm,),
        in_specs=[pl.BlockSpec((bm, _DIN), lambda i: (i, 0)),
                  pl.BlockSpec((bm, _DH), lambda i: (i, 0)),
                  pl.BlockSpec((2, bm, 16), lambda i: (0, i, 0)),
                  pl.BlockSpec((_DIN, _DH), lambda i: (0, 0)),
                  pl.BlockSpec((1, _DH), lambda i: (0, 0)),
                  pl.BlockSpec((3, _DH, _DH), lambda i: (0, 0, 0)),
                  pl.BlockSpec((3, _DH, _DH), lambda i: (0, 0, 0)),
                  pl.BlockSpec((3, _DH), lambda i: (0, 0))],
        out_specs=pl.BlockSpec((bm, _DH), lambda i: (i, 0)),
        out_shape=jax.ShapeDtypeStruct((m, _DH), jnp.float32),
    )(x, agg, deg, wfc, bfc.reshape(1, _DH), ws, wn, b)


def kernel(x_student, x_concept, x_lecture, Wfc_s, bfc_s, Wfc_c, bfc_c,
           Wfc_l, bfc_l, Ws_u, Wn_u, b_u, Ws_t, Wn_t, b_t,
           src_understands, dst_understands, src_teaches, dst_teaches):
    i32 = jnp.int32
    su = jnp.concatenate([src_understands.astype(i32),
                          jnp.zeros((_EU_PAD - _EU,), i32)])
    du = jnp.concatenate([dst_understands.astype(i32),
                          jnp.full((_EU_PAD - _EU,), _NS, i32)])
    st = jnp.concatenate([src_teaches.astype(i32),
                          jnp.zeros((_ET_PAD - _ET,), i32)])
    dt = jnp.concatenate([dst_teaches.astype(i32),
                          jnp.full((_ET_PAD - _ET,), _NL, i32)])
    du2 = jnp.concatenate([dst_understands.astype(i32),
                           jnp.full((_EU_PAD2 - _EU,), _NS, i32)])
    dt2 = jnp.concatenate([dst_teaches.astype(i32),
                           jnp.full((_ET_PAD2 - _ET,), _NL, i32)])

    gu, gt = _sc_deg(du2, dt2)
    hc = _project(x_concept, Wfc_c, bfc_c)
    fu, ft = _sc_feat(hc, su.reshape(-1, 128), du.reshape(-1, 128),
                      st.reshape(-1, 128), dt.reshape(-1, 128))

    agg_s = jnp.concatenate([fu[0, :_HALF_U], fu[1, :_HALF_U]], axis=0)
    agg_l = jnp.concatenate([ft[0, :_HALF_T], ft[1, :_HALF_T]], axis=0)
    deg_s = gu[:, :_NS]
    deg_l = gt[:, :_NL]

    hs_out = _chain(x_student, agg_s, deg_s, Wfc_s, bfc_s, Ws_u, Wn_u, b_u)
    hl_out = _chain(x_lecture, agg_l, deg_l, Wfc_l, bfc_l, Ws_t, Wn_t, b_t)
    return hs_out, hc, hl_out


# speedup vs baseline: 8.2151x; 1.5218x over previous
import functools

import jax
import jax.numpy as jnp
from jax import lax
from jax.experimental import pallas as pl
from jax.experimental.pallas import tpu as pltpu
from jax.experimental.pallas import tpu_sc as plsc

_NS, _NC, _NL = 50000, 10000, 10000
_DIN, _DH = 128, 64
_EU, _ET = 800000, 320000

_EU_PAD, _ET_PAD = 800768, 321536
_CPT_U, _CPT_T = 391, 157
_SUP = 23
_HALF_U, _HALF_T = _NS // 2, _NL // 2
_ROWS_U = 25008
_ROWS_T = 5008


_FROWS_U = 50016
_FROWS_T = 10016


def _feat_body(hc_ref, su_ref, du_ref, st_ref, dt_ref, outu_ref, outt_ref,
               acc, ssrc, sdst, rows_a, rows_b,
               semg_a, semg_b, sems_a, sems_b):
    c = lax.axis_index("c")
    s = lax.axis_index("s")
    rows = (rows_a, rows_b)
    semg = (semg_a, semg_b)
    sems = (sems_a, sems_b)
    half = 32

    def _zero_rows_a():
        def _zr(r, carry):
            for k in range(half // 16):
                rows_a[r, pl.ds(k * 16, 16)] = jnp.zeros((16,), jnp.float32)
            return carry
        lax.fori_loop(0, 128, _zr, 0)

    def _zero_region(n128, tail):
        r0 = s * (n128 * 128 + tail)
        def _za(k, carry):
            pltpu.sync_copy(rows_a, acc.at[pl.ds(r0 + k * 128, 128)])
            return carry
        lax.fori_loop(0, n128, _za, 0)
        pltpu.sync_copy(rows_a.at[pl.ds(0, tail)],
                        acc.at[pl.ds(r0 + n128 * 128, tail)])

    def _adjust_src(k, off):
        for i in range(8):
            ssrc[k, pl.ds(i * 16, 16)] = ssrc[k, pl.ds(i * 16, 16)] + off

    def _super(src2_ref, dst2_ref, chunk0, nk, off):
        pltpu.sync_copy(src2_ref.at[pl.ds(chunk0, nk)], ssrc.at[pl.ds(0, nk)])
        pltpu.sync_copy(dst2_ref.at[pl.ds(chunk0, nk)], sdst.at[pl.ds(0, nk)])
        _adjust_src(0, off)
        gd = [None] * nk
        sd = [None] * nk
        gd[0] = pltpu.async_copy(hc_ref.at[ssrc.at[0]], rows[0], semg[0])
        for j in range(nk):
            cur, nxt = j % 2, (j + 1) % 2
            if j >= 1:
                sd[j - 1].wait()
            if j + 1 < nk:
                _adjust_src(j + 1, off)
                gd[j + 1] = pltpu.async_copy(hc_ref.at[ssrc.at[j + 1]],
                                             rows[nxt], semg[nxt])
            gd[j].wait()
            sd[j] = pltpu.async_copy(rows[cur], acc.at[sdst.at[j]],
                                     sems[cur], add=True)
        sd[nk - 1].wait()

    def _phase(src2_ref, dst2_ref, n_sup, chunks_tile, off):
        base = s * chunks_tile
        def _ps(m, carry):
            _super(src2_ref, dst2_ref, base + m * _SUP, _SUP, off)
            return carry
        lax.fori_loop(0, n_sup, _ps, 0)
        rem = chunks_tile - n_sup * _SUP
        if rem:
            _super(src2_ref, dst2_ref, base + n_sup * _SUP, rem, off)

    def _writeout(out_ref, n128, tail):
        r0 = s * (n128 * 128 + tail)
        def _wo(k, carry):
            pltpu.sync_copy(acc.at[pl.ds(r0 + k * 128, 128)],
                            out_ref.at[c, pl.ds(r0 + k * 128, 128)])
            return carry
        lax.fori_loop(0, n128, _wo, 0)
        pltpu.sync_copy(acc.at[pl.ds(r0 + n128 * 128, tail)],
                        out_ref.at[c, pl.ds(r0 + n128 * 128, tail)])

    off = c * _NC
    _zero_rows_a()
    _zero_region(24, 54)
    plsc.subcore_barrier()
    _phase(su_ref, du_ref, 17, _CPT_U, off)
    plsc.subcore_barrier()
    _writeout(outu_ref, 24, 54)
    plsc.subcore_barrier()
    _zero_rows_a()
    _zero_region(4, 114)
    plsc.subcore_barrier()
    _phase(st_ref, dt_ref, 6, _CPT_T, off)
    plsc.subcore_barrier()
    _writeout(outt_ref, 4, 114)


_sc_feat = functools.partial(
    pl.kernel,
    out_type=(jax.ShapeDtypeStruct((2, _FROWS_U, 32), jnp.float32),
              jax.ShapeDtypeStruct((2, _FROWS_T, 32), jnp.float32)),
    mesh=plsc.VectorSubcoreMesh(core_axis_name="c", subcore_axis_name="s",
                                num_cores=2, num_subcores=16),
    compiler_params=pltpu.CompilerParams(use_tc_tiling_on_sc=False),
    scratch_types=[
        pltpu.VMEM_SHARED((_FROWS_U, 32), jnp.float32),
        pltpu.VMEM((_SUP, 128), jnp.int32),
        pltpu.VMEM((_SUP, 128), jnp.int32),
        pltpu.VMEM((128, 32), jnp.float32),
        pltpu.VMEM((128, 32), jnp.float32),
        pltpu.SemaphoreType.DMA, pltpu.SemaphoreType.DMA,
        pltpu.SemaphoreType.DMA, pltpu.SemaphoreType.DMA,
    ],
)(_feat_body)


_EU_PAD2, _ET_PAD2 = 802816, 323584
_DCPT_U, _DCPT_T = 196, 79
_DROWS_U = 50016
_DROWS_T = 10016


def _deg_body(du_ref, dt_ref, outu_ref, outt_ref,
              dacc, dstb, dloc, ones, zb, sem0, sem1, sem2, sem3):
    c = lax.axis_index("c")
    s = lax.axis_index("s")
    sems = (sem0, sem1, sem2, sem3)

    def _fill(buf, vec):
        def _fr(r, carry):
            buf[r, pl.ds(0, 16)] = vec
            return carry
        lax.fori_loop(0, 128, _fr, 0)

    def _zero_region(n128, tail):
        r0 = s * (n128 * 128 + tail)
        def _za(k, carry):
            pltpu.sync_copy(zb, dacc.at[pl.ds(r0 + k * 128, 128)])
            return carry
        lax.fori_loop(0, n128, _za, 0)
        pltpu.sync_copy(zb.at[pl.ds(0, tail)],
                        dacc.at[pl.ds(r0 + n128 * 128, tail)])

    def _dloc_of(ch, slot, n_dst):
        for i in range(8):
            d = dstb[pl.ds(ch * 128 + i * 16, 16)]
            ok = (d >= 0) & (d < n_dst)
            dloc[slot, pl.ds(i * 16, 16)] = jnp.where(ok, d, n_dst)

    def _scat(slot):
        return pltpu.async_copy(ones, dacc.at[dloc.at[slot]], sems[slot],
                                add=True)

    def _drain(slot):
        pltpu.make_async_copy(ones, dacc.at[dloc.at[slot]], sems[slot]).wait()

    def _phase(d_ref, n_edges_tile, n4, tail_chunks, n_dst):
        base = (c * 16 + s) * n_edges_tile
        pltpu.sync_copy(d_ref.at[pl.ds(base, n_edges_tile)],
                        dstb.at[pl.ds(0, n_edges_tile)])
        for t in range(4):
            for i in range(8):
                dloc[t, pl.ds(i * 16, 16)] = jnp.full((16,), n_dst, jnp.int32)
            _scat(t)
        def _quad(g, carry):
            for t in range(4):
                _drain(t)
                _dloc_of(4 * g + t, t, n_dst)
                _scat(t)
            return carry
        lax.fori_loop(0, n4, _quad, 0)
        for t in range(tail_chunks):
            _drain(t)
            _dloc_of(4 * n4 + t, t, n_dst)
            _scat(t)
        for t in range(4):
            _drain(t)

    def _writeout(out_ref, n128, tail):
        r0 = s * (n128 * 128 + tail)
        def _wo(k, carry):
            pltpu.sync_copy(dacc.at[pl.ds(r0 + k * 128, 128)],
                            out_ref.at[c, pl.ds(r0 + k * 128, 128)])
            return carry
        lax.fori_loop(0, n128, _wo, 0)
        pltpu.sync_copy(dacc.at[pl.ds(r0 + n128 * 128, tail)],
                        out_ref.at[c, pl.ds(r0 + n128 * 128, tail)])

    one_row = jnp.where(jnp.arange(16, dtype=jnp.int32) == 0,
                        jnp.float32(1), jnp.float32(0))
    _fill(zb, jnp.zeros((16,), jnp.float32))
    _fill(ones, one_row)
    _zero_region(24, 54)
    plsc.subcore_barrier()
    _phase(du_ref, _EU_PAD2 // 32, _DCPT_U // 4, 0, _NS)
    plsc.subcore_barrier()
    _writeout(outu_ref, 24, 54)
    plsc.subcore_barrier()
    _zero_region(4, 114)
    plsc.subcore_barrier()
    _phase(dt_ref, _ET_PAD2 // 32, _DCPT_T // 4, 3, _NL)
    plsc.subcore_barrier()
    _writeout(outt_ref, 4, 114)


_sc_deg = functools.partial(
    pl.kernel,
    out_type=(jax.ShapeDtypeStruct((2, _DROWS_U, 16), jnp.float32),
              jax.ShapeDtypeStruct((2, _DROWS_T, 16), jnp.float32)),
    mesh=plsc.VectorSubcoreMesh(core_axis_name="c", subcore_axis_name="s",
                                num_cores=2, num_subcores=16),
    compiler_params=pltpu.CompilerParams(use_tc_tiling_on_sc=False),
    scratch_types=[
        pltpu.VMEM_SHARED((_DROWS_U, 16), jnp.float32),
        pltpu.VMEM((_EU_PAD2 // 32,), jnp.int32),
        pltpu.VMEM((4, 128), jnp.int32),
        pltpu.VMEM((128, 16), jnp.float32),
        pltpu.VMEM((128, 16), jnp.float32),
        pltpu.SemaphoreType.DMA, pltpu.SemaphoreType.DMA,
        pltpu.SemaphoreType.DMA, pltpu.SemaphoreType.DMA,
    ],
)(_deg_body)


def _proj_body(x_ref, w_ref, b_ref, o_ref, o2_ref):
    h = jnp.dot(x_ref[...], w_ref[...],
                preferred_element_type=jnp.float32) + b_ref[...]
    o_ref[...] = h
    o2_ref[0] = h[:, :32]
    o2_ref[1] = h[:, 32:]


def _project(x, w, b):
    m, bm = x.shape[0], 2000
    return pl.pallas_call(
        _proj_body,
        grid=(m // bm,),
        in_specs=[pl.BlockSpec((bm, _DIN), lambda i: (i, 0)),
                  pl.BlockSpec((_DIN, _DH), lambda i: (0, 0)),
                  pl.BlockSpec((1, _DH), lambda i: (0, 0))],
        out_specs=[pl.BlockSpec((bm, _DH), lambda i: (i, 0)),
                   pl.BlockSpec((2, bm, 32), lambda i: (0, i, 0))],
        out_shape=[jax.ShapeDtypeStruct((m, _DH), jnp.float32),
                   jax.ShapeDtypeStruct((2, m, 32), jnp.float32)],
    )(x, w, b.reshape(1, _DH))


def _chain_body(x_ref, g_ref, d_ref, wfc_ref, bfc_ref, ws_ref, wn_ref, b_ref,
                o_ref):
    h = jnp.dot(x_ref[...], wfc_ref[...],
                preferred_element_type=jnp.float32) + bfc_ref[...]
    deg = d_ref[0, :, 0:1] + d_ref[1, :, 0:1]
    nbr = g_ref[...] / jnp.maximum(deg, 1.0)
    for i in range(3):
        z = (jnp.dot(h, ws_ref[i], preferred_element_type=jnp.float32)
             + jnp.dot(nbr, wn_ref[i], preferred_element_type=jnp.float32)
             + b_ref[i])
        h = jnp.maximum(z, 0.0) if i < 2 else z
    o_ref[...] = h


def _chain(x, agg, deg, wfc, bfc, ws, wn, b):
    m, bm = x.shape[0], 2000
    return pl.pallas_call(
        _chain_body,
        grid=(m // bm,),
        in_specs=[pl.BlockSpec((bm, _DIN), lambda i: (i, 0)),
                  pl.BlockSpec((bm, _DH), lambda i: (i, 0)),
                  pl.BlockSpec((2, bm, 16), lambda i: (0, i, 0)),
                  pl.BlockSpec((_DIN, _DH), lambda i: (0, 0)),
                  pl.BlockSpec((1, _DH), lambda i: (0, 0)),
                  pl.BlockSpec((3, _DH, _DH), lambda i: (0, 0, 0)),
                  pl.BlockSpec((3, _DH, _DH), lambda i: (0, 0, 0)),
                  pl.BlockSpec((3, _DH), lambda i: (0, 0))],
        out_specs=pl.BlockSpec((bm, _DH), lambda i: (i, 0)),
        out_shape=jax.ShapeDtypeStruct((m, _DH), jnp.float32),
    )(x, agg, deg, wfc, bfc.reshape(1, _DH), ws, wn, b)


def kernel(x_student, x_concept, x_lecture, Wfc_s, bfc_s, Wfc_c, bfc_c,
           Wfc_l, bfc_l, Ws_u, Wn_u, b_u, Ws_t, Wn_t, b_t,
           src_understands, dst_understands, src_teaches, dst_teaches):
    i32 = jnp.int32
    su = jnp.concatenate([src_understands.astype(i32),
                          jnp.zeros((_EU_PAD - _EU,), i32)])
    du = jnp.concatenate([dst_understands.astype(i32),
                          jnp.full((_EU_PAD - _EU,), _NS, i32)])
    st = jnp.concatenate([src_teaches.astype(i32),
                          jnp.zeros((_ET_PAD - _ET,), i32)])
    dt = jnp.concatenate([dst_teaches.astype(i32),
                          jnp.full((_ET_PAD - _ET,), _NL, i32)])
    du2 = jnp.concatenate([dst_understands.astype(i32),
                           jnp.full((_EU_PAD2 - _EU,), _NS, i32)])
    dt2 = jnp.concatenate([dst_teaches.astype(i32),
                           jnp.full((_ET_PAD2 - _ET,), _NL, i32)])

    gu, gt = _sc_deg(du2, dt2)
    hc, hc2 = _project(x_concept, Wfc_c, bfc_c)
    fu, ft = _sc_feat(hc2.reshape(2 * _NC, 32), su.reshape(-1, 128),
                      du.reshape(-1, 128), st.reshape(-1, 128),
                      dt.reshape(-1, 128))

    agg_s = jnp.concatenate([fu[0, :_NS], fu[1, :_NS]], axis=1)
    agg_l = jnp.concatenate([ft[0, :_NL], ft[1, :_NL]], axis=1)
    deg_s = gu[:, :_NS]
    deg_l = gt[:, :_NL]

    hs_out = _chain(x_student, agg_s, deg_s, Wfc_s, bfc_s, Ws_u, Wn_u, b_u)
    hl_out = _chain(x_lecture, agg_l, deg_l, Wfc_l, bfc_l, Ws_t, Wn_t, b_t)
    return hs_out, hc, hl_out


# speedup vs baseline: 9.6564x; 1.1755x over previous
import functools

import jax
import jax.numpy as jnp
from jax import lax
from jax.experimental import pallas as pl
from jax.experimental.pallas import tpu as pltpu
from jax.experimental.pallas import tpu_sc as plsc

_NS, _NC, _NL = 50000, 10000, 10000
_DIN, _DH = 128, 64
_EU, _ET = 800000, 320000

_EU_PAD, _ET_PAD = 800768, 321536
_CPT_U, _CPT_T = 391, 157
_SUP = 23
_HALF_U, _HALF_T = _NS // 2, _NL // 2
_ROWS_U = 25008
_ROWS_T = 5008


_FROWS_U = 50016
_FROWS_T = 10016


def _feat_body(hc_ref, su_ref, du_ref, st_ref, dt_ref, outu_ref, outt_ref,
               acc, ssrc, sdst, rows_a, rows_b,
               semg_a, semg_b, sems_a, sems_b):
    c = lax.axis_index("c")
    s = lax.axis_index("s")
    rows = (rows_a, rows_b)
    semg = (semg_a, semg_b)
    sems = (sems_a, sems_b)
    half = 32

    def _zero_rows_a():
        def _zr(r, carry):
            for k in range(half // 16):
                rows_a[r, pl.ds(k * 16, 16)] = jnp.zeros((16,), jnp.float32)
            return carry
        lax.fori_loop(0, 128, _zr, 0)

    def _zero_region(n128, tail):
        r0 = s * (n128 * 128 + tail)
        def _za(k, carry):
            pltpu.sync_copy(rows_a, acc.at[pl.ds(r0 + k * 128, 128)])
            return carry
        lax.fori_loop(0, n128, _za, 0)
        pltpu.sync_copy(rows_a.at[pl.ds(0, tail)],
                        acc.at[pl.ds(r0 + n128 * 128, tail)])

    def _adjust_src(k, off):
        for i in range(8):
            ssrc[k, pl.ds(i * 16, 16)] = ssrc[k, pl.ds(i * 16, 16)] + off

    def _super(src2_ref, dst2_ref, chunk0, nk, off):
        pltpu.sync_copy(src2_ref.at[pl.ds(chunk0, nk)], ssrc.at[pl.ds(0, nk)])
        pltpu.sync_copy(dst2_ref.at[pl.ds(chunk0, nk)], sdst.at[pl.ds(0, nk)])
        _adjust_src(0, off)
        gd = [None] * nk
        sd = [None] * nk
        gd[0] = pltpu.async_copy(hc_ref.at[ssrc.at[0]], rows[0], semg[0])
        for j in range(nk):
            cur, nxt = j % 2, (j + 1) % 2
            if j >= 1:
                sd[j - 1].wait()
            if j + 1 < nk:
                _adjust_src(j + 1, off)
                gd[j + 1] = pltpu.async_copy(hc_ref.at[ssrc.at[j + 1]],
                                             rows[nxt], semg[nxt])
            gd[j].wait()
            sd[j] = pltpu.async_copy(rows[cur], acc.at[sdst.at[j]],
                                     sems[cur], add=True)
        sd[nk - 1].wait()

    def _phase(src2_ref, dst2_ref, n_sup, chunks_tile, off):
        base = s * chunks_tile
        def _ps(m, carry):
            _super(src2_ref, dst2_ref, base + m * _SUP, _SUP, off)
            return carry
        lax.fori_loop(0, n_sup, _ps, 0)
        rem = chunks_tile - n_sup * _SUP
        if rem:
            _super(src2_ref, dst2_ref, base + n_sup * _SUP, rem, off)

    def _writeout(out_ref, n128, tail, tail_last):
        r0 = s * (n128 * 128 + tail)
        def _wo(k, carry):
            pltpu.sync_copy(acc.at[pl.ds(r0 + k * 128, 128)],
                            out_ref.at[c, pl.ds(r0 + k * 128, 128)])
            return carry
        lax.fori_loop(0, n128, _wo, 0)
        @pl.when(s < 15)
        def _t():
            pltpu.sync_copy(acc.at[pl.ds(r0 + n128 * 128, tail)],
                            out_ref.at[c, pl.ds(r0 + n128 * 128, tail)])
        @pl.when(s == 15)
        def _tl():
            pltpu.sync_copy(acc.at[pl.ds(r0 + n128 * 128, tail_last)],
                            out_ref.at[c, pl.ds(r0 + n128 * 128, tail_last)])

    off = c * _NC
    _zero_rows_a()
    _zero_region(24, 54)
    plsc.subcore_barrier()
    _phase(su_ref, du_ref, 17, _CPT_U, off)
    plsc.subcore_barrier()
    _writeout(outu_ref, 24, 54, 38)
    plsc.subcore_barrier()
    _zero_rows_a()
    _zero_region(4, 114)
    plsc.subcore_barrier()
    _phase(st_ref, dt_ref, 6, _CPT_T, off)
    plsc.subcore_barrier()
    _writeout(outt_ref, 4, 114, 98)


_sc_feat = functools.partial(
    pl.kernel,
    out_type=(jax.ShapeDtypeStruct((2, _NS, 32), jnp.float32),
              jax.ShapeDtypeStruct((2, _NL, 32), jnp.float32)),
    mesh=plsc.VectorSubcoreMesh(core_axis_name="c", subcore_axis_name="s",
                                num_cores=2, num_subcores=16),
    compiler_params=pltpu.CompilerParams(use_tc_tiling_on_sc=False),
    scratch_types=[
        pltpu.VMEM_SHARED((_FROWS_U, 32), jnp.float32),
        pltpu.VMEM((_SUP, 128), jnp.int32),
        pltpu.VMEM((_SUP, 128), jnp.int32),
        pltpu.VMEM((128, 32), jnp.float32),
        pltpu.VMEM((128, 32), jnp.float32),
        pltpu.SemaphoreType.DMA, pltpu.SemaphoreType.DMA,
        pltpu.SemaphoreType.DMA, pltpu.SemaphoreType.DMA,
    ],
)(_feat_body)


_EU_PAD2, _ET_PAD2 = 802816, 323584
_DCPT_U, _DCPT_T = 196, 79
_DROWS_U = 50016
_DROWS_T = 10016


def _deg_body(du_ref, dt_ref, outu_ref, outt_ref,
              dacc, dstb, ones, zb, sem0, sem1, sem2, sem3):
    c = lax.axis_index("c")
    s = lax.axis_index("s")
    sems = (sem0, sem1, sem2, sem3)

    def _fill(buf, vec):
        def _fr(r, carry):
            buf[r, pl.ds(0, 16)] = vec
            return carry
        lax.fori_loop(0, 128, _fr, 0)

    def _zero_region(n128, tail):
        r0 = s * (n128 * 128 + tail)
        def _za(k, carry):
            pltpu.sync_copy(zb, dacc.at[pl.ds(r0 + k * 128, 128)])
            return carry
        lax.fori_loop(0, n128, _za, 0)
        pltpu.sync_copy(zb.at[pl.ds(0, tail)],
                        dacc.at[pl.ds(r0 + n128 * 128, tail)])

    def _scat(ch, slot):
        return pltpu.async_copy(ones, dacc.at[dstb.at[ch]], sems[slot],
                                add=True)

    def _drain(slot):
        pltpu.make_async_copy(ones, dacc.at[dstb.at[0]], sems[slot]).wait()

    def _phase(d2_ref, n_chunks, n4, tail_chunks):
        base = (c * 16 + s) * n_chunks
        pltpu.sync_copy(d2_ref.at[pl.ds(base, n_chunks)],
                        dstb.at[pl.ds(0, n_chunks)])
        for t in range(4):
            _scat(t, t)
        def _quad(g, carry):
            for t in range(4):
                _drain(t)
                _scat(4 * g + 4 + t, t)
            return carry
        lax.fori_loop(0, n4, _quad, 0)
        for t in range(tail_chunks):
            _drain(t)
            _scat(4 * n4 + 4 + t, t)
        for t in range(4):
            _drain(t)

    def _writeout(out_ref, n128, tail, tail_last):
        r0 = s * (n128 * 128 + tail)
        def _wo(k, carry):
            pltpu.sync_copy(dacc.at[pl.ds(r0 + k * 128, 128)],
                            out_ref.at[c, pl.ds(r0 + k * 128, 128)])
            return carry
        lax.fori_loop(0, n128, _wo, 0)
        @pl.when(s < 15)
        def _t():
            pltpu.sync_copy(dacc.at[pl.ds(r0 + n128 * 128, tail)],
                            out_ref.at[c, pl.ds(r0 + n128 * 128, tail)])
        @pl.when(s == 15)
        def _tl():
            pltpu.sync_copy(dacc.at[pl.ds(r0 + n128 * 128, tail_last)],
                            out_ref.at[c, pl.ds(r0 + n128 * 128, tail_last)])

    one_row = jnp.where(jnp.arange(16, dtype=jnp.int32) == 0,
                        jnp.float32(1), jnp.float32(0))
    _fill(zb, jnp.zeros((16,), jnp.float32))
    _fill(ones, one_row)
    _zero_region(24, 54)
    plsc.subcore_barrier()
    _phase(du_ref, _DCPT_U, (_DCPT_U - 4) // 4, 0)
    plsc.subcore_barrier()
    _writeout(outu_ref, 24, 54, 38)
    plsc.subcore_barrier()
    _zero_region(4, 114)
    plsc.subcore_barrier()
    _phase(dt_ref, _DCPT_T, (_DCPT_T - 7) // 4, 3)
    plsc.subcore_barrier()
    _writeout(outt_ref, 4, 114, 98)


_sc_deg = functools.partial(
    pl.kernel,
    out_type=(jax.ShapeDtypeStruct((2, _NS, 16), jnp.float32),
              jax.ShapeDtypeStruct((2, _NL, 16), jnp.float32)),
    mesh=plsc.VectorSubcoreMesh(core_axis_name="c", subcore_axis_name="s",
                                num_cores=2, num_subcores=16),
    compiler_params=pltpu.CompilerParams(use_tc_tiling_on_sc=False),
    scratch_types=[
        pltpu.VMEM_SHARED((_DROWS_U, 16), jnp.float32),
        pltpu.VMEM((_DCPT_U, 128), jnp.int32),
        pltpu.VMEM((128, 16), jnp.float32),
        pltpu.VMEM((128, 16), jnp.float32),
        pltpu.SemaphoreType.DMA, pltpu.SemaphoreType.DMA,
        pltpu.SemaphoreType.DMA, pltpu.SemaphoreType.DMA,
    ],
)(_deg_body)


def _proj_body(x_ref, w_ref, b_ref, o_ref, o2_ref):
    h = jnp.dot(x_ref[...], w_ref[...],
                preferred_element_type=jnp.float32) + b_ref[...]
    o_ref[...] = h
    o2_ref[0] = h[:, :32]
    o2_ref[1] = h[:, 32:]


def _project(x, w, b):
    m, bm = x.shape[0], 2000
    return pl.pallas_call(
        _proj_body,
        grid=(m // bm,),
        in_specs=[pl.BlockSpec((bm, _DIN), lambda i: (i, 0)),
                  pl.BlockSpec((_DIN, _DH), lambda i: (0, 0)),
                  pl.BlockSpec((1, _DH), lambda i: (0, 0))],
        out_specs=[pl.BlockSpec((bm, _DH), lambda i: (i, 0)),
                   pl.BlockSpec((2, bm, 32), lambda i: (0, i, 0))],
        out_shape=[jax.ShapeDtypeStruct((m, _DH), jnp.float32),
                   jax.ShapeDtypeStruct((2, m, 32), jnp.float32)],
    )(x, w, b.reshape(1, _DH))


def _chain_body(x_ref, g_ref, d_ref, wfc_ref, bfc_ref, ws_ref, wn_ref, b_ref,
                o_ref):
    h = jnp.dot(x_ref[...], wfc_ref[...],
                preferred_element_type=jnp.float32) + bfc_ref[...]
    deg = d_ref[0, :, 0:1] + d_ref[1, :, 0:1]
    agg = jnp.concatenate([g_ref[0], g_ref[1]], axis=1)
    nbr = agg / jnp.maximum(deg, 1.0)
    for i in range(3):
        z = (jnp.dot(h, ws_ref[i], preferred_element_type=jnp.float32)
             + jnp.dot(nbr, wn_ref[i], preferred_element_type=jnp.float32)
             + b_ref[i])
        h = jnp.maximum(z, 0.0) if i < 2 else z
    o_ref[...] = h


def _chain(x, agg, deg, wfc, bfc, ws, wn, b):
    m, bm = x.shape[0], 2000
    return pl.pallas_call(
        _chain_body,
        grid=(m // bm,),
        in_specs=[pl.BlockSpec((bm, _DIN), lambda i: (i, 0)),
                  pl.BlockSpec((2, bm, 32), lambda i: (0, i, 0)),
                  pl.BlockSpec((2, bm, 16), lambda i: (0, i, 0)),
                  pl.BlockSpec((_DIN, _DH), lambda i: (0, 0)),
                  pl.BlockSpec((1, _DH), lambda i: (0, 0)),
                  pl.BlockSpec((3, _DH, _DH), lambda i: (0, 0, 0)),
                  pl.BlockSpec((3, _DH, _DH), lambda i: (0, 0, 0)),
                  pl.BlockSpec((3, _DH), lambda i: (0, 0))],
        out_specs=pl.BlockSpec((bm, _DH), lambda i: (i, 0)),
        out_shape=jax.ShapeDtypeStruct((m, _DH), jnp.float32),
    )(x, agg, deg, wfc, bfc.reshape(1, _DH), ws, wn, b)


def kernel(x_student, x_concept, x_lecture, Wfc_s, bfc_s, Wfc_c, bfc_c,
           Wfc_l, bfc_l, Ws_u, Wn_u, b_u, Ws_t, Wn_t, b_t,
           src_understands, dst_understands, src_teaches, dst_teaches):
    i32 = jnp.int32
    su = jnp.concatenate([src_understands.astype(i32),
                          jnp.zeros((_EU_PAD - _EU,), i32)])
    du = jnp.concatenate([dst_understands.astype(i32),
                          jnp.full((_EU_PAD - _EU,), _NS, i32)])
    st = jnp.concatenate([src_teaches.astype(i32),
                          jnp.zeros((_ET_PAD - _ET,), i32)])
    dt = jnp.concatenate([dst_teaches.astype(i32),
                          jnp.full((_ET_PAD - _ET,), _NL, i32)])
    du2 = jnp.concatenate([dst_understands.astype(i32),
                           jnp.full((_EU_PAD2 - _EU,), _NS, i32)])
    dt2 = jnp.concatenate([dst_teaches.astype(i32),
                           jnp.full((_ET_PAD2 - _ET,), _NL, i32)])

    gu, gt = _sc_deg(du2.reshape(-1, 128), dt2.reshape(-1, 128))
    hc, hc2 = _project(x_concept, Wfc_c, bfc_c)
    fu, ft = _sc_feat(hc2.reshape(2 * _NC, 32), su.reshape(-1, 128),
                      du.reshape(-1, 128), st.reshape(-1, 128),
                      dt.reshape(-1, 128))

    hs_out = _chain(x_student, fu, gu, Wfc_s, bfc_s, Ws_u, Wn_u, b_u)
    hl_out = _chain(x_lecture, ft, gt, Wfc_l, bfc_l, Ws_t, Wn_t, b_t)
    return hs_out, hc, hl_out


# speedup vs baseline: 10.3047x; 1.0671x over previous
import functools

import jax
import jax.numpy as jnp
from jax import lax
from jax.experimental import pallas as pl
from jax.experimental.pallas import tpu as pltpu
from jax.experimental.pallas import tpu_sc as plsc

_NS, _NC, _NL = 50000, 10000, 10000
_DIN, _DH = 128, 64
_EU, _ET = 800000, 320000

_EU_PAD, _ET_PAD = 800768, 321536
_CPT_U, _CPT_T = 391, 157
_SUP = 23
_HALF_U, _HALF_T = _NS // 2, _NL // 2
_ROWS_U = 25008
_ROWS_T = 5008


_FROWS_U = 50016
_FROWS_T = 10016


def _feat_body(hc_ref, su_ref, du_ref, st_ref, dt_ref, outu_ref, outt_ref,
               acc, ssrc, sdst, rows_a, rows_b, rows_c, rows_d,
               semg_a, semg_b, semg_c, semg_d,
               sems_a, sems_b, sems_c, sems_d):
    c = lax.axis_index("c")
    s = lax.axis_index("s")
    rows = (rows_a, rows_b, rows_c, rows_d)
    semg = (semg_a, semg_b, semg_c, semg_d)
    sems = (sems_a, sems_b, sems_c, sems_d)
    half = 32

    def _zero_rows_a():
        def _zr(r, carry):
            for k in range(half // 16):
                rows_a[r, pl.ds(k * 16, 16)] = jnp.zeros((16,), jnp.float32)
            return carry
        lax.fori_loop(0, 128, _zr, 0)

    def _zero_region(n128, tail):
        r0 = s * (n128 * 128 + tail)
        def _za(k, carry):
            pltpu.sync_copy(rows_a, acc.at[pl.ds(r0 + k * 128, 128)])
            return carry
        lax.fori_loop(0, n128, _za, 0)
        pltpu.sync_copy(rows_a.at[pl.ds(0, tail)],
                        acc.at[pl.ds(r0 + n128 * 128, tail)])

    def _adjust_src(k, off):
        for i in range(8):
            ssrc[k, pl.ds(i * 16, 16)] = ssrc[k, pl.ds(i * 16, 16)] + off

    def _super(src2_ref, dst2_ref, chunk0, nk, off):
        pltpu.sync_copy(src2_ref.at[pl.ds(chunk0, nk)], ssrc.at[pl.ds(0, nk)])
        pltpu.sync_copy(dst2_ref.at[pl.ds(chunk0, nk)], sdst.at[pl.ds(0, nk)])
        _adjust_src(0, off)
        gd = [None] * nk
        sd = [None] * nk
        gd[0] = pltpu.async_copy(hc_ref.at[ssrc.at[0]], rows[0], semg[0])
        for j in range(nk):
            cur, nxt = j % 4, (j + 1) % 4
            if j >= 3:
                sd[j - 3].wait()
            if j + 1 < nk:
                _adjust_src(j + 1, off)
                gd[j + 1] = pltpu.async_copy(hc_ref.at[ssrc.at[j + 1]],
                                             rows[nxt], semg[nxt])
            gd[j].wait()
            sd[j] = pltpu.async_copy(rows[cur], acc.at[sdst.at[j]],
                                     sems[cur], add=True)
        for j in range(max(nk - 3, 0), nk):
            sd[j].wait()

    def _phase(src2_ref, dst2_ref, n_sup, chunks_tile, off):
        base = s * chunks_tile
        def _ps(m, carry):
            _super(src2_ref, dst2_ref, base + m * _SUP, _SUP, off)
            return carry
        lax.fori_loop(0, n_sup, _ps, 0)
        rem = chunks_tile - n_sup * _SUP
        if rem:
            _super(src2_ref, dst2_ref, base + n_sup * _SUP, rem, off)

    def _writeout(out_ref, n128, tail, tail_last):
        r0 = s * (n128 * 128 + tail)
        def _wo(k, carry):
            pltpu.sync_copy(acc.at[pl.ds(r0 + k * 128, 128)],
                            out_ref.at[c, pl.ds(r0 + k * 128, 128)])
            return carry
        lax.fori_loop(0, n128, _wo, 0)
        @pl.when(s < 15)
        def _t():
            pltpu.sync_copy(acc.at[pl.ds(r0 + n128 * 128, tail)],
                            out_ref.at[c, pl.ds(r0 + n128 * 128, tail)])
        @pl.when(s == 15)
        def _tl():
            pltpu.sync_copy(acc.at[pl.ds(r0 + n128 * 128, tail_last)],
                            out_ref.at[c, pl.ds(r0 + n128 * 128, tail_last)])

    off = c * _NC
    _zero_rows_a()
    _zero_region(24, 54)
    plsc.subcore_barrier()
    _phase(su_ref, du_ref, 17, _CPT_U, off)
    plsc.subcore_barrier()
    _writeout(outu_ref, 24, 54, 38)
    plsc.subcore_barrier()
    _zero_rows_a()
    _zero_region(4, 114)
    plsc.subcore_barrier()
    _phase(st_ref, dt_ref, 6, _CPT_T, off)
    plsc.subcore_barrier()
    _writeout(outt_ref, 4, 114, 98)


_sc_feat = functools.partial(
    pl.kernel,
    out_type=(jax.ShapeDtypeStruct((2, _NS, 32), jnp.float32),
              jax.ShapeDtypeStruct((2, _NL, 32), jnp.float32)),
    mesh=plsc.VectorSubcoreMesh(core_axis_name="c", subcore_axis_name="s",
                                num_cores=2, num_subcores=16),
    compiler_params=pltpu.CompilerParams(use_tc_tiling_on_sc=False),
    scratch_types=[
        pltpu.VMEM_SHARED((_FROWS_U, 32), jnp.float32),
        pltpu.VMEM((_SUP, 128), jnp.int32),
        pltpu.VMEM((_SUP, 128), jnp.int32),
        pltpu.VMEM((128, 32), jnp.float32),
        pltpu.VMEM((128, 32), jnp.float32),
        pltpu.VMEM((128, 32), jnp.float32),
        pltpu.VMEM((128, 32), jnp.float32),
        pltpu.SemaphoreType.DMA, pltpu.SemaphoreType.DMA,
        pltpu.SemaphoreType.DMA, pltpu.SemaphoreType.DMA,
        pltpu.SemaphoreType.DMA, pltpu.SemaphoreType.DMA,
        pltpu.SemaphoreType.DMA, pltpu.SemaphoreType.DMA,
    ],
)(_feat_body)


_EU_PAD2, _ET_PAD2 = 802816, 323584
_DCPT_U, _DCPT_T = 196, 79
_DROWS_U = 50016
_DROWS_T = 10016


def _deg_body(du_ref, dt_ref, outu_ref, outt_ref,
              dacc, dstb, ones, zb, sem0, sem1, sem2, sem3):
    c = lax.axis_index("c")
    s = lax.axis_index("s")
    sems = (sem0, sem1, sem2, sem3)

    def _fill(buf, vec):
        def _fr(r, carry):
            buf[r, pl.ds(0, 16)] = vec
            return carry
        lax.fori_loop(0, 128, _fr, 0)

    def _zero_region(n128, tail):
        r0 = s * (n128 * 128 + tail)
        def _za(k, carry):
            pltpu.sync_copy(zb, dacc.at[pl.ds(r0 + k * 128, 128)])
            return carry
        lax.fori_loop(0, n128, _za, 0)
        pltpu.sync_copy(zb.at[pl.ds(0, tail)],
                        dacc.at[pl.ds(r0 + n128 * 128, tail)])

    def _scat(ch, slot):
        return pltpu.async_copy(ones, dacc.at[dstb.at[ch]], sems[slot],
                                add=True)

    def _drain(slot):
        pltpu.make_async_copy(ones, dacc.at[dstb.at[0]], sems[slot]).wait()

    def _phase(d2_ref, n_chunks, n4, tail_chunks):
        base = (c * 16 + s) * n_chunks
        pltpu.sync_copy(d2_ref.at[pl.ds(base, n_chunks)],
                        dstb.at[pl.ds(0, n_chunks)])
        for t in range(4):
            _scat(t, t)
        def _quad(g, carry):
            for t in range(4):
                _drain(t)
                _scat(4 * g + 4 + t, t)
            return carry
        lax.fori_loop(0, n4, _quad, 0)
        for t in range(tail_chunks):
            _drain(t)
            _scat(4 * n4 + 4 + t, t)
        for t in range(4):
            _drain(t)

    def _writeout(out_ref, n128, tail, tail_last):
        r0 = s * (n128 * 128 + tail)
        def _wo(k, carry):
            pltpu.sync_copy(dacc.at[pl.ds(r0 + k * 128, 128)],
                            out_ref.at[c, pl.ds(r0 + k * 128, 128)])
            return carry
        lax.fori_loop(0, n128, _wo, 0)
        @pl.when(s < 15)
        def _t():
            pltpu.sync_copy(dacc.at[pl.ds(r0 + n128 * 128, tail)],
                            out_ref.at[c, pl.ds(r0 + n128 * 128, tail)])
        @pl.when(s == 15)
        def _tl():
            pltpu.sync_copy(dacc.at[pl.ds(r0 + n128 * 128, tail_last)],
                            out_ref.at[c, pl.ds(r0 + n128 * 128, tail_last)])

    one_row = jnp.where(jnp.arange(16, dtype=jnp.int32) == 0,
                        jnp.float32(1), jnp.float32(0))
    _fill(zb, jnp.zeros((16,), jnp.float32))
    _fill(ones, one_row)
    _zero_region(24, 54)
    plsc.subcore_barrier()
    _phase(du_ref, _DCPT_U, (_DCPT_U - 4) // 4, 0)
    plsc.subcore_barrier()
    _writeout(outu_ref, 24, 54, 38)
    plsc.subcore_barrier()
    _zero_region(4, 114)
    plsc.subcore_barrier()
    _phase(dt_ref, _DCPT_T, (_DCPT_T - 7) // 4, 3)
    plsc.subcore_barrier()
    _writeout(outt_ref, 4, 114, 98)


_sc_deg = functools.partial(
    pl.kernel,
    out_type=(jax.ShapeDtypeStruct((2, _NS, 16), jnp.float32),
              jax.ShapeDtypeStruct((2, _NL, 16), jnp.float32)),
    mesh=plsc.VectorSubcoreMesh(core_axis_name="c", subcore_axis_name="s",
                                num_cores=2, num_subcores=16),
    compiler_params=pltpu.CompilerParams(use_tc_tiling_on_sc=False),
    scratch_types=[
        pltpu.VMEM_SHARED((_DROWS_U, 16), jnp.float32),
        pltpu.VMEM((_DCPT_U, 128), jnp.int32),
        pltpu.VMEM((128, 16), jnp.float32),
        pltpu.VMEM((128, 16), jnp.float32),
        pltpu.SemaphoreType.DMA, pltpu.SemaphoreType.DMA,
        pltpu.SemaphoreType.DMA, pltpu.SemaphoreType.DMA,
    ],
)(_deg_body)


def _proj1_body(x_ref, w_ref, b_ref, o_ref):
    o_ref[...] = jnp.dot(x_ref[...], w_ref[...],
                         preferred_element_type=jnp.float32) + b_ref[...]


def _project1(x, w, b):
    m, bm = x.shape[0], 2000
    return pl.pallas_call(
        _proj1_body,
        grid=(m // bm,),
        in_specs=[pl.BlockSpec((bm, _DIN), lambda i: (i, 0)),
                  pl.BlockSpec((_DIN, _DH), lambda i: (0, 0)),
                  pl.BlockSpec((1, _DH), lambda i: (0, 0))],
        out_specs=pl.BlockSpec((bm, _DH), lambda i: (i, 0)),
        out_shape=jax.ShapeDtypeStruct((m, _DH), jnp.float32),
    )(x, w, b.reshape(1, _DH))


def _proj_body(x_ref, w_ref, b_ref, o_ref, o2_ref):
    h = jnp.dot(x_ref[...], w_ref[...],
                preferred_element_type=jnp.float32) + b_ref[...]
    o_ref[...] = h
    o2_ref[0] = h[:, :32]
    o2_ref[1] = h[:, 32:]


def _project(x, w, b):
    m, bm = x.shape[0], 2000
    return pl.pallas_call(
        _proj_body,
        grid=(m // bm,),
        in_specs=[pl.BlockSpec((bm, _DIN), lambda i: (i, 0)),
                  pl.BlockSpec((_DIN, _DH), lambda i: (0, 0)),
                  pl.BlockSpec((1, _DH), lambda i: (0, 0))],
        out_specs=[pl.BlockSpec((bm, _DH), lambda i: (i, 0)),
                   pl.BlockSpec((2, bm, 32), lambda i: (0, i, 0))],
        out_shape=[jax.ShapeDtypeStruct((m, _DH), jnp.float32),
                   jax.ShapeDtypeStruct((2, m, 32), jnp.float32)],
    )(x, w, b.reshape(1, _DH))


def _chain_body(h_ref, g_ref, d_ref, ws_ref, wn_ref, b_ref, o_ref):
    h = h_ref[...]
    deg = d_ref[0, :, 0:1] + d_ref[1, :, 0:1]
    agg = jnp.concatenate([g_ref[0], g_ref[1]], axis=1)
    nbr = agg / jnp.maximum(deg, 1.0)
    for i in range(3):
        z = (jnp.dot(h, ws_ref[i], preferred_element_type=jnp.float32)
             + jnp.dot(nbr, wn_ref[i], preferred_element_type=jnp.float32)
             + b_ref[i])
        h = jnp.maximum(z, 0.0) if i < 2 else z
    o_ref[...] = h


def _chain(h0, agg, deg, ws, wn, b):
    m, bm = h0.shape[0], 2000
    return pl.pallas_call(
        _chain_body,
        grid=(m // bm,),
        in_specs=[pl.BlockSpec((bm, _DH), lambda i: (i, 0)),
                  pl.BlockSpec((2, bm, 32), lambda i: (0, i, 0)),
                  pl.BlockSpec((2, bm, 16), lambda i: (0, i, 0)),
                  pl.BlockSpec((3, _DH, _DH), lambda i: (0, 0, 0)),
                  pl.BlockSpec((3, _DH, _DH), lambda i: (0, 0, 0)),
                  pl.BlockSpec((3, _DH), lambda i: (0, 0))],
        out_specs=pl.BlockSpec((bm, _DH), lambda i: (i, 0)),
        out_shape=jax.ShapeDtypeStruct((m, _DH), jnp.float32),
    )(h0, agg, deg, ws, wn, b)


def kernel(x_student, x_concept, x_lecture, Wfc_s, bfc_s, Wfc_c, bfc_c,
           Wfc_l, bfc_l, Ws_u, Wn_u, b_u, Ws_t, Wn_t, b_t,
           src_understands, dst_understands, src_teaches, dst_teaches):
    i32 = jnp.int32
    su = jnp.concatenate([src_understands.astype(i32),
                          jnp.zeros((_EU_PAD - _EU,), i32)])
    du = jnp.concatenate([dst_understands.astype(i32),
                          jnp.full((_EU_PAD - _EU,), _NS, i32)])
    st = jnp.concatenate([src_teaches.astype(i32),
                          jnp.zeros((_ET_PAD - _ET,), i32)])
    dt = jnp.concatenate([dst_teaches.astype(i32),
                          jnp.full((_ET_PAD - _ET,), _NL, i32)])
    du2 = jnp.concatenate([dst_understands.astype(i32),
                           jnp.full((_EU_PAD2 - _EU,), _NS, i32)])
    dt2 = jnp.concatenate([dst_teaches.astype(i32),
                           jnp.full((_ET_PAD2 - _ET,), _NL, i32)])

    hc, hc2 = _project(x_concept, Wfc_c, bfc_c)
    gu, gt = _sc_deg(du2.reshape(-1, 128), dt2.reshape(-1, 128))
    fu, ft = _sc_feat(hc2.reshape(2 * _NC, 32), su.reshape(-1, 128),
                      du.reshape(-1, 128), st.reshape(-1, 128),
                      dt.reshape(-1, 128))
    hs0 = _project1(x_student, Wfc_s, bfc_s)
    hl0 = _project1(x_lecture, Wfc_l, bfc_l)

    hs_out = _chain(hs0, fu, gu, Ws_u, Wn_u, b_u)
    hl_out = _chain(hl0, ft, gt, Ws_t, Wn_t, b_t)
    return hs_out, hc, hl_out


# speedup vs baseline: 10.7116x; 1.0395x over previous
import functools

import jax
import jax.numpy as jnp
from jax import lax
from jax.experimental import pallas as pl
from jax.experimental.pallas import tpu as pltpu
from jax.experimental.pallas import tpu_sc as plsc

_NS, _NC, _NL = 50000, 10000, 10000
_DIN, _DH = 128, 64
_EU, _ET = 800000, 320000

_EU_PAD, _ET_PAD = 800768, 322048
_CPT_U, _CPT_T = 391, 157
_SUP = 23
_HALF_U, _HALF_T = _NS // 2, _NL // 2
_ROWS_U = 25008
_ROWS_T = 5008


_FROWS_U = 50016
_FROWS_T = 10016


def _feat_body(hc_ref, su_ref, du_ref, st_ref, dt_ref, outu_ref, outt_ref,
               acc, ssrc_a, ssrc_b, sdst_a, sdst_b,
               rows_a, rows_b, rows_c, rows_d,
               semg_a, semg_b, semg_c, semg_d,
               sems_a, sems_b, sems_c, sems_d, semi_a, semi_b):
    c = lax.axis_index("c")
    s = lax.axis_index("s")
    ssrc_r = (ssrc_a, ssrc_b)
    sdst_r = (sdst_a, sdst_b)
    semi = (semi_a, semi_b)
    rows = (rows_a, rows_b, rows_c, rows_d)
    semg = (semg_a, semg_b, semg_c, semg_d)
    sems = (sems_a, sems_b, sems_c, sems_d)
    half = 32

    def _zero_rows_a():
        def _zr(r, carry):
            for k in range(half // 16):
                rows_a[r, pl.ds(k * 16, 16)] = jnp.zeros((16,), jnp.float32)
            return carry
        lax.fori_loop(0, 128, _zr, 0)

    def _zero_region(n128, tail):
        r0 = s * (n128 * 128 + tail)
        def _za(k, carry):
            pltpu.sync_copy(rows_a, acc.at[pl.ds(r0 + k * 128, 128)])
            return carry
        lax.fori_loop(0, n128, _za, 0)
        pltpu.sync_copy(rows_a.at[pl.ds(0, tail)],
                        acc.at[pl.ds(r0 + n128 * 128, tail)])

    def _adjust_src(ssrc, k, off):
        for i in range(8):
            ssrc[k, pl.ds(i * 16, 16)] = ssrc[k, pl.ds(i * 16, 16)] + off

    def _fire_idx(src2_ref, dst2_ref, chunk0, slot):
        pltpu.async_copy(src2_ref.at[pl.ds(chunk0, _SUP)], ssrc_r[slot],
                         semi[slot])
        pltpu.async_copy(dst2_ref.at[pl.ds(chunk0, _SUP)], sdst_r[slot],
                         semi[slot])

    def _drain_idx(src2_ref, dst2_ref, slot):
        pltpu.make_async_copy(src2_ref.at[pl.ds(0, _SUP)], ssrc_r[slot],
                              semi[slot]).wait()
        pltpu.make_async_copy(dst2_ref.at[pl.ds(0, _SUP)], sdst_r[slot],
                              semi[slot]).wait()

    def _super(slot, nk, off):
        ssrc = ssrc_r[slot]
        sdst = sdst_r[slot]
        _adjust_src(ssrc, 0, off)
        gd = [None] * nk
        sd = [None] * nk
        gd[0] = pltpu.async_copy(hc_ref.at[ssrc.at[0]], rows[0], semg[0])
        for j in range(nk):
            cur, nxt = j % 4, (j + 1) % 4
            if j >= 3:
                sd[j - 3].wait()
            if j + 1 < nk:
                _adjust_src(ssrc, j + 1, off)
                gd[j + 1] = pltpu.async_copy(hc_ref.at[ssrc.at[j + 1]],
                                             rows[nxt], semg[nxt])
            gd[j].wait()
            sd[j] = pltpu.async_copy(rows[cur], acc.at[sdst.at[j]],
                                     sems[cur], add=True)
        for j in range(max(nk - 3, 0), nk):
            sd[j].wait()

    def _phase(src2_ref, dst2_ref, n_sup, chunks_tile, off, tail_nk):
        base = s * chunks_tile
        n_tot = n_sup + (1 if tail_nk else 0)
        _fire_idx(src2_ref, dst2_ref, base, 0)
        def _pp(g, carry):
            m0 = 2 * g
            _drain_idx(src2_ref, dst2_ref, 0)
            _fire_idx(src2_ref, dst2_ref, base + (m0 + 1) * _SUP, 1)
            _super(0, _SUP, off)
            _drain_idx(src2_ref, dst2_ref, 1)
            @pl.when(m0 + 2 < n_tot)
            def _pf():
                _fire_idx(src2_ref, dst2_ref, base + (m0 + 2) * _SUP, 0)
            _super(1, _SUP, off)
            return carry
        lax.fori_loop(0, n_sup // 2, _pp, 0)
        if n_sup % 2:
            _drain_idx(src2_ref, dst2_ref, 0)
            if tail_nk:
                _fire_idx(src2_ref, dst2_ref, base + n_sup * _SUP, 1)
            _super(0, _SUP, off)
            if tail_nk:
                _drain_idx(src2_ref, dst2_ref, 1)
                _super(1, tail_nk, off)
        elif tail_nk:
            _drain_idx(src2_ref, dst2_ref, 0)
            _super(0, tail_nk, off)

    def _writeout(out_ref, n128, tail, tail_last):
        r0 = s * (n128 * 128 + tail)
        def _wo(k, carry):
            pltpu.sync_copy(acc.at[pl.ds(r0 + k * 128, 128)],
                            out_ref.at[c, pl.ds(r0 + k * 128, 128)])
            return carry
        lax.fori_loop(0, n128, _wo, 0)
        @pl.when(s < 15)
        def _t():
            pltpu.sync_copy(acc.at[pl.ds(r0 + n128 * 128, tail)],
                            out_ref.at[c, pl.ds(r0 + n128 * 128, tail)])
        @pl.when(s == 15)
        def _tl():
            pltpu.sync_copy(acc.at[pl.ds(r0 + n128 * 128, tail_last)],
                            out_ref.at[c, pl.ds(r0 + n128 * 128, tail_last)])

    off = c * _NC
    _zero_rows_a()
    _zero_region(24, 54)
    plsc.subcore_barrier()
    _phase(su_ref, du_ref, 17, _CPT_U, off, 0)
    plsc.subcore_barrier()
    _writeout(outu_ref, 24, 54, 38)
    plsc.subcore_barrier()
    _zero_rows_a()
    _zero_region(4, 114)
    plsc.subcore_barrier()
    _phase(st_ref, dt_ref, 6, _CPT_T, off, 19)
    plsc.subcore_barrier()
    _writeout(outt_ref, 4, 114, 98)


_sc_feat = functools.partial(
    pl.kernel,
    out_type=(jax.ShapeDtypeStruct((2, _NS, 32), jnp.float32),
              jax.ShapeDtypeStruct((2, _NL, 32), jnp.float32)),
    mesh=plsc.VectorSubcoreMesh(core_axis_name="c", subcore_axis_name="s",
                                num_cores=2, num_subcores=16),
    compiler_params=pltpu.CompilerParams(use_tc_tiling_on_sc=False),
    scratch_types=[
        pltpu.VMEM_SHARED((_FROWS_U, 32), jnp.float32),
        pltpu.VMEM((_SUP, 128), jnp.int32),
        pltpu.VMEM((_SUP, 128), jnp.int32),
        pltpu.VMEM((_SUP, 128), jnp.int32),
        pltpu.VMEM((_SUP, 128), jnp.int32),
        pltpu.VMEM((128, 32), jnp.float32),
        pltpu.VMEM((128, 32), jnp.float32),
        pltpu.VMEM((128, 32), jnp.float32),
        pltpu.VMEM((128, 32), jnp.float32),
        pltpu.SemaphoreType.DMA, pltpu.SemaphoreType.DMA,
        pltpu.SemaphoreType.DMA, pltpu.SemaphoreType.DMA,
        pltpu.SemaphoreType.DMA, pltpu.SemaphoreType.DMA,
        pltpu.SemaphoreType.DMA, pltpu.SemaphoreType.DMA,
        pltpu.SemaphoreType.DMA, pltpu.SemaphoreType.DMA,
    ],
)(_feat_body)


_EU_PAD2, _ET_PAD2 = 802816, 323584
_DCPT_U, _DCPT_T = 196, 79
_DROWS_U = 50016
_DROWS_T = 10016


def _deg_body(du_ref, dt_ref, outu_ref, outt_ref,
              dacc, dstb, ones, zb, sem0, sem1, sem2, sem3):
    c = lax.axis_index("c")
    s = lax.axis_index("s")
    sems = (sem0, sem1, sem2, sem3)

    def _fill(buf, vec):
        def _fr(r, carry):
            buf[r, pl.ds(0, 16)] = vec
            return carry
        lax.fori_loop(0, 128, _fr, 0)

    def _zero_region(n128, tail):
        r0 = s * (n128 * 128 + tail)
        def _za(k, carry):
            pltpu.sync_copy(zb, dacc.at[pl.ds(r0 + k * 128, 128)])
            return carry
        lax.fori_loop(0, n128, _za, 0)
        pltpu.sync_copy(zb.at[pl.ds(0, tail)],
                        dacc.at[pl.ds(r0 + n128 * 128, tail)])

    def _scat(ch, slot):
        return pltpu.async_copy(ones, dacc.at[dstb.at[ch]], sems[slot],
                                add=True)

    def _drain(slot):
        pltpu.make_async_copy(ones, dacc.at[dstb.at[0]], sems[slot]).wait()

    def _phase(d2_ref, n_chunks, n4, tail_chunks):
        base = (c * 16 + s) * n_chunks
        pltpu.sync_copy(d2_ref.at[pl.ds(base, n_chunks)],
                        dstb.at[pl.ds(0, n_chunks)])
        for t in range(4):
            _scat(t, t)
        def _quad(g, carry):
            for t in range(4):
                _drain(t)
                _scat(4 * g + 4 + t, t)
            return carry
        lax.fori_loop(0, n4, _quad, 0)
        for t in range(tail_chunks):
            _drain(t)
            _scat(4 * n4 + 4 + t, t)
        for t in range(4):
            _drain(t)

    def _writeout(out_ref, n128, tail, tail_last):
        r0 = s * (n128 * 128 + tail)
        def _wo(k, carry):
            pltpu.sync_copy(dacc.at[pl.ds(r0 + k * 128, 128)],
                            out_ref.at[c, pl.ds(r0 + k * 128, 128)])
            return carry
        lax.fori_loop(0, n128, _wo, 0)
        @pl.when(s < 15)
        def _t():
            pltpu.sync_copy(dacc.at[pl.ds(r0 + n128 * 128, tail)],
                            out_ref.at[c, pl.ds(r0 + n128 * 128, tail)])
        @pl.when(s == 15)
        def _tl():
            pltpu.sync_copy(dacc.at[pl.ds(r0 + n128 * 128, tail_last)],
                            out_ref.at[c, pl.ds(r0 + n128 * 128, tail_last)])

    one_row = jnp.where(jnp.arange(16, dtype=jnp.int32) == 0,
                        jnp.float32(1), jnp.float32(0))
    _fill(zb, jnp.zeros((16,), jnp.float32))
    _fill(ones, one_row)
    _zero_region(24, 54)
    plsc.subcore_barrier()
    _phase(du_ref, _DCPT_U, (_DCPT_U - 4) // 4, 0)
    plsc.subcore_barrier()
    _writeout(outu_ref, 24, 54, 38)
    plsc.subcore_barrier()
    _zero_region(4, 114)
    plsc.subcore_barrier()
    _phase(dt_ref, _DCPT_T, (_DCPT_T - 7) // 4, 3)
    plsc.subcore_barrier()
    _writeout(outt_ref, 4, 114, 98)


_sc_deg = functools.partial(
    pl.kernel,
    out_type=(jax.ShapeDtypeStruct((2, _NS, 16), jnp.float32),
              jax.ShapeDtypeStruct((2, _NL, 16), jnp.float32)),
    mesh=plsc.VectorSubcoreMesh(core_axis_name="c", subcore_axis_name="s",
                                num_cores=2, num_subcores=16),
    compiler_params=pltpu.CompilerParams(use_tc_tiling_on_sc=False),
    scratch_types=[
        pltpu.VMEM_SHARED((_DROWS_U, 16), jnp.float32),
        pltpu.VMEM((_DCPT_U, 128), jnp.int32),
        pltpu.VMEM((128, 16), jnp.float32),
        pltpu.VMEM((128, 16), jnp.float32),
        pltpu.SemaphoreType.DMA, pltpu.SemaphoreType.DMA,
        pltpu.SemaphoreType.DMA, pltpu.SemaphoreType.DMA,
    ],
)(_deg_body)


def _proj1_body(x_ref, w_ref, b_ref, o_ref):
    o_ref[...] = jnp.dot(x_ref[...], w_ref[...],
                         preferred_element_type=jnp.float32) + b_ref[...]


def _project1(x, w, b):
    m, bm = x.shape[0], 2000
    return pl.pallas_call(
        _proj1_body,
        grid=(m // bm,),
        in_specs=[pl.BlockSpec((bm, _DIN), lambda i: (i, 0)),
                  pl.BlockSpec((_DIN, _DH), lambda i: (0, 0)),
                  pl.BlockSpec((1, _DH), lambda i: (0, 0))],
        out_specs=pl.BlockSpec((bm, _DH), lambda i: (i, 0)),
        out_shape=jax.ShapeDtypeStruct((m, _DH), jnp.float32),
    )(x, w, b.reshape(1, _DH))


def _proj_body(x_ref, w_ref, b_ref, o_ref, o2_ref):
    h = jnp.dot(x_ref[...], w_ref[...],
                preferred_element_type=jnp.float32) + b_ref[...]
    o_ref[...] = h
    o2_ref[0] = h[:, :32]
    o2_ref[1] = h[:, 32:]


def _project(x, w, b):
    m, bm = x.shape[0], 2000
    return pl.pallas_call(
        _proj_body,
        grid=(m // bm,),
        in_specs=[pl.BlockSpec((bm, _DIN), lambda i: (i, 0)),
                  pl.BlockSpec((_DIN, _DH), lambda i: (0, 0)),
                  pl.BlockSpec((1, _DH), lambda i: (0, 0))],
        out_specs=[pl.BlockSpec((bm, _DH), lambda i: (i, 0)),
                   pl.BlockSpec((2, bm, 32), lambda i: (0, i, 0))],
        out_shape=[jax.ShapeDtypeStruct((m, _DH), jnp.float32),
                   jax.ShapeDtypeStruct((2, m, 32), jnp.float32)],
    )(x, w, b.reshape(1, _DH))


def _chain_body(h_ref, g_ref, d_ref, ws_ref, wn_ref, b_ref, o_ref):
    h = h_ref[...]
    deg = d_ref[0, :, 0:1] + d_ref[1, :, 0:1]
    agg = jnp.concatenate([g_ref[0], g_ref[1]], axis=1)
    nbr = agg / jnp.maximum(deg, 1.0)
    for i in range(3):
        z = (jnp.dot(h, ws_ref[i], preferred_element_type=jnp.float32)
             + jnp.dot(nbr, wn_ref[i], preferred_element_type=jnp.float32)
             + b_ref[i])
        h = jnp.maximum(z, 0.0) if i < 2 else z
    o_ref[...] = h


def _chain(h0, agg, deg, ws, wn, b):
    m, bm = h0.shape[0], 2000
    return pl.pallas_call(
        _chain_body,
        grid=(m // bm,),
        in_specs=[pl.BlockSpec((bm, _DH), lambda i: (i, 0)),
                  pl.BlockSpec((2, bm, 32), lambda i: (0, i, 0)),
                  pl.BlockSpec((2, bm, 16), lambda i: (0, i, 0)),
                  pl.BlockSpec((3, _DH, _DH), lambda i: (0, 0, 0)),
                  pl.BlockSpec((3, _DH, _DH), lambda i: (0, 0, 0)),
                  pl.BlockSpec((3, _DH), lambda i: (0, 0))],
        out_specs=pl.BlockSpec((bm, _DH), lambda i: (i, 0)),
        out_shape=jax.ShapeDtypeStruct((m, _DH), jnp.float32),
    )(h0, agg, deg, ws, wn, b)


def kernel(x_student, x_concept, x_lecture, Wfc_s, bfc_s, Wfc_c, bfc_c,
           Wfc_l, bfc_l, Ws_u, Wn_u, b_u, Ws_t, Wn_t, b_t,
           src_understands, dst_understands, src_teaches, dst_teaches):
    i32 = jnp.int32
    su = jnp.concatenate([src_understands.astype(i32),
                          jnp.zeros((_EU_PAD - _EU,), i32)])
    du = jnp.concatenate([dst_understands.astype(i32),
                          jnp.full((_EU_PAD - _EU,), _NS, i32)])
    st = jnp.concatenate([src_teaches.astype(i32),
                          jnp.zeros((_ET_PAD - _ET,), i32)])
    dt = jnp.concatenate([dst_teaches.astype(i32),
                          jnp.full((_ET_PAD - _ET,), _NL, i32)])
    du2 = jnp.concatenate([dst_understands.astype(i32),
                           jnp.full((_EU_PAD2 - _EU,), _NS, i32)])
    dt2 = jnp.concatenate([dst_teaches.astype(i32),
                           jnp.full((_ET_PAD2 - _ET,), _NL, i32)])

    hc, hc2 = _project(x_concept, Wfc_c, bfc_c)
    gu, gt = _sc_deg(du2.reshape(-1, 128), dt2.reshape(-1, 128))
    fu, ft = _sc_feat(hc2.reshape(2 * _NC, 32), su.reshape(-1, 128),
                      du.reshape(-1, 128), st.reshape(-1, 128),
                      dt.reshape(-1, 128))
    hs0 = _project1(x_student, Wfc_s, bfc_s)
    hl0 = _project1(x_lecture, Wfc_l, bfc_l)

    hs_out = _chain(hs0, fu, gu, Ws_u, Wn_u, b_u)
    hl_out = _chain(hl0, ft, gt, Ws_t, Wn_t, b_t)
    return hs_out, hc, hl_out
